# Initial kernel scaffold; baseline (speedup 1.0000x reference)
import jax
import jax.numpy as jnp
from jax.experimental import pallas as pl


def kernel(x, g1, b1, Wqkv, Wo, g2, b2, W1, bb1, gf1, bf1, W2, bb2, gf2, bf2):
    raise NotImplementedError("write your pallas kernel here")


# speedup vs baseline: 1.1234x; 1.1234x over previous
import functools

import jax
import jax.numpy as jnp
from jax.experimental import pallas as pl
from jax.experimental.pallas import tpu as pltpu

TAU = 2.0
THR = 1.0
NWIN = (2, 4, 4)
K_TOP = 4
N_HEADS = 8
EPS = 1e-5


def _bnlif_kernel(x_ref, g_ref, b_ref, s_ref, *, nck):
    T, N, cb = x_ref.shape
    ck = N // nck
    acc = jnp.zeros((1, cb), jnp.float32)
    acc2 = jnp.zeros((1, cb), jnp.float32)
    for c in range(nck):
        xc = x_ref[:, c * ck:(c + 1) * ck, :]
        acc = acc + jnp.sum(xc, axis=(0, 1), keepdims=False)[None]
        acc2 = acc2 + jnp.sum(xc * xc, axis=(0, 1), keepdims=False)[None]
    cnt = float(T * N)
    m = acc / cnt
    var = acc2 / cnt - m * m
    scale = jax.lax.rsqrt(var + EPS) * g_ref[0]
    shift = b_ref[0] - m * scale
    for c in range(nck):
        sl = pl.ds(c * ck, ck)
        v = jnp.zeros((ck, cb), jnp.float32)
        for t in range(T):
            xn = x_ref[t, sl, :] * scale + shift
            v = v + (xn - v) / TAU
            sp = (v - THR >= 0).astype(jnp.float32)
            s_ref[t, sl, :] = sp
            v = v - sp * THR


def _add_kernel(x_ref, a_ref, o_ref):
    o_ref[...] = x_ref[...] + a_ref[...]


def _route_kernel(s_ref, wqkv_ref, q_ref, k_ref, v_ref, idx_ref, *, P, w, C):
    s = s_ref[0]
    qkv = jnp.dot(s, wqkv_ref[...], preferred_element_type=jnp.float32)
    q = qkv[:, :C]
    k = qkv[:, C:2 * C]
    v = qkv[:, 2 * C:]
    q_ref[0] = q
    k_ref[0] = k
    v_ref[0] = v
    qm = jnp.mean(q.reshape(P, w, C), axis=1)
    km = jnp.mean(k.reshape(P, w, C), axis=1)
    aff = jax.lax.dot_general(qm, km, (((1,), (1,)), ((), ())),
                              preferred_element_type=jnp.float32)
    col = jax.lax.broadcasted_iota(jnp.int32, (P, P), 1)
    cur = aff
    for j in range(K_TOP):
        mx = jnp.max(cur, axis=1, keepdims=True)
        cand = jnp.where(cur == mx, col, jnp.int32(2 ** 30))
        idxj = jnp.min(cand, axis=1)
        idx_ref[0, j, :] = idxj
        cur = jnp.where(col == idxj[:, None], -jnp.inf, cur)


def _attn_kernel(idx_sref, q_ref, k_ref, v_ref, wo_ref, o_ref, *, w, C):
    b = pl.program_id(0)
    p = pl.program_id(1)
    dh = C // N_HEADS
    q = q_ref[0, 0]
    ks = []
    vs = []
    for j in range(K_TOP):
        widx = idx_sref[b, j, p]
        ks.append(k_ref[0, pl.ds(widx * w, w), :])
        vs.append(v_ref[0, pl.ds(widx * w, w), :])
    kg = jnp.concatenate(ks, axis=0)
    vg = jnp.concatenate(vs, axis=0)
    scale = dh ** -0.5
    outs = []
    for h in range(N_HEADS):
        qh = q[:, h * dh:(h + 1) * dh]
        kh = kg[:, h * dh:(h + 1) * dh]
        vh = vg[:, h * dh:(h + 1) * dh]
        logits = jax.lax.dot_general(qh, kh, (((1,), (1,)), ((), ())),
                                     preferred_element_type=jnp.float32) * scale
        logits = logits - jnp.max(logits, axis=1, keepdims=True)
        e = jnp.exp(logits)
        att = e / jnp.sum(e, axis=1, keepdims=True)
        outs.append(jnp.dot(att, vh, preferred_element_type=jnp.float32))
    o = jnp.concatenate(outs, axis=1)
    o_ref[0, 0] = jnp.dot(o, wo_ref[...], preferred_element_type=jnp.float32)


def _ffn1_kernel(s_ref, w_ref, bb_ref, h_ref, st_ref, acc_ref, *, M):
    i = pl.program_id(0)
    h = jnp.dot(s_ref[...], w_ref[...],
                preferred_element_type=jnp.float32) + bb_ref[...]
    h_ref[...] = h

    @pl.when(i == 0)
    def _():
        acc_ref[...] = jnp.zeros_like(acc_ref)

    acc_ref[0:1, :] += jnp.sum(h, axis=0, keepdims=True)
    acc_ref[1:2, :] += jnp.sum(h * h, axis=0, keepdims=True)

    @pl.when(i == pl.num_programs(0) - 1)
    def _():
        m = acc_ref[0:1, :] / M
        ex2 = acc_ref[1:2, :] / M
        st_ref[...] = jnp.concatenate([m, ex2 - m * m], axis=0)


def _ffn2_kernel(h_ref, st1_ref, g_ref, b_ref, w_ref, bb_ref,
                 o_ref, st_ref, acc_ref, *, M):
    i = pl.program_id(0)
    h = h_ref[...]
    hn = (h - st1_ref[0:1, :]) * jax.lax.rsqrt(st1_ref[1:2, :] + EPS)
    hn = hn * g_ref[...] + b_ref[...]
    hg = 0.5 * hn * (1.0 + jax.lax.erf(hn * (2.0 ** -0.5)))
    o = jnp.dot(hg, w_ref[...],
                preferred_element_type=jnp.float32) + bb_ref[...]
    o_ref[...] = o

    @pl.when(i == 0)
    def _():
        acc_ref[...] = jnp.zeros_like(acc_ref)

    acc_ref[0:1, :] += jnp.sum(o, axis=0, keepdims=True)
    acc_ref[1:2, :] += jnp.sum(o * o, axis=0, keepdims=True)

    @pl.when(i == pl.num_programs(0) - 1)
    def _():
        m = acc_ref[0:1, :] / M
        ex2 = acc_ref[1:2, :] / M
        st_ref[...] = jnp.concatenate([m, ex2 - m * m], axis=0)


def _final_kernel(o_ref, x2_ref, st_ref, g_ref, b_ref, out_ref):
    o = o_ref[...]
    on = (o - st_ref[0:1, :]) * jax.lax.rsqrt(st_ref[1:2, :] + EPS)
    out_ref[...] = x2_ref[...] + on * g_ref[...] + b_ref[...]


def kernel(x, g1, b1, Wqkv, Wo, g2, b2, W1, bb1, gf1, bf1, W2, bb2, gf2, bf2):
    T, B, C, Lt, Lh, Lw = x.shape
    nt, nh, nw = NWIN
    wt, wh, ww = Lt // nt, Lh // nh, Lw // nw
    P = nt * nh * nw
    w = wt * wh * ww
    N = B * P * w
    M = T * N
    Cf = 4 * C
    cb = 128
    ncb = C // cb

    xw = x.transpose(0, 1, 3, 4, 5, 2)
    xw = xw.reshape(T, B, nt, wt, nh, wh, nw, ww, C)
    xw = xw.transpose(0, 1, 2, 4, 6, 3, 5, 7, 8)
    Xw = xw.reshape(T, N, C)

    g1r = g1.reshape(1, 1, C)
    b1r = b1.reshape(1, 1, C)
    g2r = g2.reshape(1, 1, C)
    b2r = b2.reshape(1, 1, C)

    s1 = pl.pallas_call(
        functools.partial(_bnlif_kernel, nck=8),
        grid=(ncb,),
        in_specs=[
            pl.BlockSpec((T, N, cb), lambda i: (0, 0, i)),
            pl.BlockSpec((1, 1, cb), lambda i: (0, 0, i)),
            pl.BlockSpec((1, 1, cb), lambda i: (0, 0, i)),
        ],
        out_specs=pl.BlockSpec((T, N, cb), lambda i: (0, 0, i)),
        out_shape=jax.ShapeDtypeStruct((T, N, C), jnp.float32),
    )(Xw, g1r, b1r)

    TB = T * B
    Pw = P * w
    s1b = s1.reshape(T, B, Pw, C).reshape(TB, Pw, C)
    q, k, v, idx = pl.pallas_call(
        functools.partial(_route_kernel, P=P, w=w, C=C),
        grid=(TB,),
        in_specs=[
            pl.BlockSpec((1, Pw, C), lambda i: (i, 0, 0)),
            pl.BlockSpec((C, 3 * C), lambda i: (0, 0)),
        ],
        out_specs=[
            pl.BlockSpec((1, Pw, C), lambda i: (i, 0, 0)),
            pl.BlockSpec((1, Pw, C), lambda i: (i, 0, 0)),
            pl.BlockSpec((1, Pw, C), lambda i: (i, 0, 0)),
            pl.BlockSpec((1, K_TOP, P), lambda i: (i, 0, 0)),
        ],
        out_shape=[
            jax.ShapeDtypeStruct((TB, Pw, C), jnp.float32),
            jax.ShapeDtypeStruct((TB, Pw, C), jnp.float32),
            jax.ShapeDtypeStruct((TB, Pw, C), jnp.float32),
            jax.ShapeDtypeStruct((TB, K_TOP, P), jnp.int32),
        ],
    )(s1b, Wqkv)

    q4 = q.reshape(TB, P, w, C)
    ao = pl.pallas_call(
        functools.partial(_attn_kernel, w=w, C=C),
        grid_spec=pltpu.PrefetchScalarGridSpec(
            num_scalar_prefetch=1,
            grid=(TB, P),
            in_specs=[
                pl.BlockSpec((1, 1, w, C), lambda b, p, idx_ref: (b, p, 0, 0)),
                pl.BlockSpec((1, Pw, C), lambda b, p, idx_ref: (b, 0, 0)),
                pl.BlockSpec((1, Pw, C), lambda b, p, idx_ref: (b, 0, 0)),
                pl.BlockSpec((C, C), lambda b, p, idx_ref: (0, 0)),
            ],
            out_specs=pl.BlockSpec((1, 1, w, C),
                                   lambda b, p, idx_ref: (b, p, 0, 0)),
        ),
        out_shape=jax.ShapeDtypeStruct((TB, P, w, C), jnp.float32),
    )(idx, q4, k, v, Wo)

    aow = ao.reshape(TB, Pw, C).reshape(T, N, C)
    nb2 = 8
    x2 = pl.pallas_call(
        _add_kernel,
        grid=(nb2,),
        in_specs=[
            pl.BlockSpec((T, N // nb2, C), lambda i: (0, i, 0)),
            pl.BlockSpec((T, N // nb2, C), lambda i: (0, i, 0)),
        ],
        out_specs=pl.BlockSpec((T, N // nb2, C), lambda i: (0, i, 0)),
        out_shape=jax.ShapeDtypeStruct((T, N, C), jnp.float32),
    )(Xw, aow)
    s2 = pl.pallas_call(
        functools.partial(_bnlif_kernel, nck=8),
        grid=(ncb,),
        in_specs=[
            pl.BlockSpec((T, N, cb), lambda i: (0, 0, i)),
            pl.BlockSpec((1, 1, cb), lambda i: (0, 0, i)),
            pl.BlockSpec((1, 1, cb), lambda i: (0, 0, i)),
        ],
        out_specs=pl.BlockSpec((T, N, cb), lambda i: (0, 0, i)),
        out_shape=jax.ShapeDtypeStruct((T, N, C), jnp.float32),
    )(x2, g2r, b2r)

    s2v = s2.reshape(M, C)
    x2v = x2.reshape(M, C)
    mb = 512
    nmb = M // mb
    W1T = W1.T
    W2T = W2.T

    h, st1 = pl.pallas_call(
        functools.partial(_ffn1_kernel, M=float(M)),
        grid=(nmb,),
        in_specs=[
            pl.BlockSpec((mb, C), lambda i: (i, 0)),
            pl.BlockSpec((C, Cf), lambda i: (0, 0)),
            pl.BlockSpec((1, Cf), lambda i: (0, 0)),
        ],
        out_specs=[
            pl.BlockSpec((mb, Cf), lambda i: (i, 0)),
            pl.BlockSpec((2, Cf), lambda i: (0, 0)),
        ],
        out_shape=[
            jax.ShapeDtypeStruct((M, Cf), jnp.float32),
            jax.ShapeDtypeStruct((2, Cf), jnp.float32),
        ],
        scratch_shapes=[pltpu.VMEM((2, Cf), jnp.float32)],
    )(s2v, W1T, bb1.reshape(1, Cf))

    o2, st2 = pl.pallas_call(
        functools.partial(_ffn2_kernel, M=float(M)),
        grid=(nmb,),
        in_specs=[
            pl.BlockSpec((mb, Cf), lambda i: (i, 0)),
            pl.BlockSpec((2, Cf), lambda i: (0, 0)),
            pl.BlockSpec((1, Cf), lambda i: (0, 0)),
            pl.BlockSpec((1, Cf), lambda i: (0, 0)),
            pl.BlockSpec((Cf, C), lambda i: (0, 0)),
            pl.BlockSpec((1, C), lambda i: (0, 0)),
        ],
        out_specs=[
            pl.BlockSpec((mb, C), lambda i: (i, 0)),
            pl.BlockSpec((2, C), lambda i: (0, 0)),
        ],
        out_shape=[
            jax.ShapeDtypeStruct((M, C), jnp.float32),
            jax.ShapeDtypeStruct((2, C), jnp.float32),
        ],
        scratch_shapes=[pltpu.VMEM((2, C), jnp.float32)],
    )(h, st1, gf1.reshape(1, Cf), bf1.reshape(1, Cf), W2T,
      bb2.reshape(1, C))

    outv = pl.pallas_call(
        _final_kernel,
        grid=(nmb,),
        in_specs=[
            pl.BlockSpec((mb, C), lambda i: (i, 0)),
            pl.BlockSpec((mb, C), lambda i: (i, 0)),
            pl.BlockSpec((2, C), lambda i: (0, 0)),
            pl.BlockSpec((1, C), lambda i: (0, 0)),
            pl.BlockSpec((1, C), lambda i: (0, 0)),
        ],
        out_specs=pl.BlockSpec((mb, C), lambda i: (i, 0)),
        out_shape=jax.ShapeDtypeStruct((M, C), jnp.float32),
    )(o2, x2v, st2, gf2.reshape(1, C), bf2.reshape(1, C))

    out = outv.reshape(T, B, nt, nh, nw, wt, wh, ww, C)
    out = out.transpose(0, 1, 2, 5, 3, 6, 4, 7, 8)
    out = out.reshape(T, B, Lt, Lh, Lw, C)
    return out.transpose(0, 1, 5, 2, 3, 4)


# speedup vs baseline: 1.1594x; 1.0321x over previous
import functools

import jax
import jax.numpy as jnp
from jax.experimental import pallas as pl
from jax.experimental.pallas import tpu as pltpu

TAU = 2.0
THR = 1.0
NWIN = (2, 4, 4)
K_TOP = 4
N_HEADS = 8
EPS = 1e-5


def _bnlif_kernel(x_ref, g_ref, b_ref, s_ref, *, nck):
    T, N, cb = x_ref.shape
    ck = N // nck
    acc = jnp.zeros((1, cb), jnp.float32)
    acc2 = jnp.zeros((1, cb), jnp.float32)
    for c in range(nck):
        xc = x_ref[:, c * ck:(c + 1) * ck, :]
        acc = acc + jnp.sum(xc, axis=(0, 1), keepdims=False)[None]
        acc2 = acc2 + jnp.sum(xc * xc, axis=(0, 1), keepdims=False)[None]
    cnt = float(T * N)
    m = acc / cnt
    var = acc2 / cnt - m * m
    scale = jax.lax.rsqrt(var + EPS) * g_ref[0]
    shift = b_ref[0] - m * scale
    for c in range(nck):
        sl = pl.ds(c * ck, ck)
        v = jnp.zeros((ck, cb), jnp.float32)
        for t in range(T):
            xn = x_ref[t, sl, :] * scale + shift
            v = v + (xn - v) / TAU
            sp = (v - THR >= 0).astype(jnp.float32)
            s_ref[t, sl, :] = sp
            v = v - sp * THR


def _add_kernel(x_ref, a_ref, o_ref):
    o_ref[...] = x_ref[...] + a_ref[...]


def _route_kernel(s_ref, wqkv_ref, qkv_ref, idx_ref, *, P, w, C):
    s = s_ref[0]
    qkv = jnp.dot(s, wqkv_ref[...], preferred_element_type=jnp.float32)
    qkv_ref[0] = qkv
    q = qkv[:, :C]
    k = qkv[:, C:2 * C]
    qm = jnp.mean(q.reshape(P, w, C), axis=1)
    km = jnp.mean(k.reshape(P, w, C), axis=1)
    aff = jax.lax.dot_general(qm, km, (((1,), (1,)), ((), ())),
                              preferred_element_type=jnp.float32)
    col = jax.lax.broadcasted_iota(jnp.int32, (P, P), 1)
    cur = aff
    for j in range(K_TOP):
        mx = jnp.max(cur, axis=1, keepdims=True)
        cand = jnp.where(cur == mx, col, jnp.int32(2 ** 30))
        idxj = jnp.min(cand, axis=1)
        idx_ref[0, j, :] = idxj
        cur = jnp.where(col == idxj[:, None], -jnp.inf, cur)


def _attn_kernel(idx_sref, qkv_ref, wo_ref, o_ref, *, P, w, C):
    b = pl.program_id(0)
    dh = C // N_HEADS
    scale = dh ** -0.5
    wo = wo_ref[...]

    def body(p, carry):
        q = qkv_ref[0, pl.ds(p * w, w), 0:C]
        ks = []
        vs = []
        for j in range(K_TOP):
            wi = idx_sref[b, j, p]
            ks.append(qkv_ref[0, pl.ds(wi * w, w), C:2 * C])
            vs.append(qkv_ref[0, pl.ds(wi * w, w), 2 * C:3 * C])
        kg = jnp.concatenate(ks, axis=0)
        vg = jnp.concatenate(vs, axis=0)
        outs = []
        for h in range(N_HEADS):
            qh = q[:, h * dh:(h + 1) * dh]
            kh = kg[:, h * dh:(h + 1) * dh]
            vh = vg[:, h * dh:(h + 1) * dh]
            logits = jax.lax.dot_general(
                qh, kh, (((1,), (1,)), ((), ())),
                preferred_element_type=jnp.float32) * scale
            logits = logits - jnp.max(logits, axis=1, keepdims=True)
            e = jnp.exp(logits)
            att = e / jnp.sum(e, axis=1, keepdims=True)
            outs.append(jnp.dot(att, vh, preferred_element_type=jnp.float32))
        o = jnp.concatenate(outs, axis=1)
        o_ref[0, pl.ds(p * w, w), :] = jnp.dot(
            o, wo, preferred_element_type=jnp.float32)
        return carry

    jax.lax.fori_loop(0, P, body, 0)


def _ffnstats_kernel(s_ref, w1_ref, bb_ref, st_ref, cs_ref, gram_ref, *, M):
    i = pl.program_id(0)
    s = s_ref[...]

    @pl.when(i == 0)
    def _():
        cs_ref[...] = jnp.zeros_like(cs_ref)
        gram_ref[...] = jnp.zeros_like(gram_ref)

    cs_ref[...] += jnp.sum(s, axis=0, keepdims=True)
    sb = s.astype(jnp.bfloat16)
    gram_ref[...] += jax.lax.dot_general(
        sb, sb, (((0,), (0,)), ((), ())), preferred_element_type=jnp.float32)

    @pl.when(i == pl.num_programs(0) - 1)
    def _():
        wmat = w1_ref[...]
        sw = jnp.dot(gram_ref[...], wmat, preferred_element_type=jnp.float32)
        diag = jnp.sum(wmat * sw, axis=0, keepdims=True)
        cw = jnp.dot(cs_ref[...], wmat, preferred_element_type=jnp.float32)
        bb = bb_ref[...]
        sumh = cw + M * bb
        sumh2 = diag + 2.0 * bb * cw + M * bb * bb
        m = sumh / M
        var = sumh2 / M - m * m
        st_ref[...] = jnp.concatenate([m, var], axis=0)


def _ffn_kernel(s_ref, w1_ref, bb1_ref, st1_ref, g_ref, b_ref,
                w2_ref, bb2_ref, o_ref, st_ref, acc_ref, *, M):
    i = pl.program_id(0)
    sb = s_ref[...].astype(jnp.bfloat16)
    h = jnp.dot(sb, w1_ref[...],
                preferred_element_type=jnp.float32) + bb1_ref[...]
    hn = (h - st1_ref[0:1, :]) * jax.lax.rsqrt(st1_ref[1:2, :] + EPS)
    hn = hn * g_ref[...] + b_ref[...]
    hg = 0.5 * hn * (1.0 + jax.lax.erf(hn * (2.0 ** -0.5)))
    o = jnp.dot(hg.astype(jnp.bfloat16), w2_ref[...],
                preferred_element_type=jnp.float32) + bb2_ref[...]
    o_ref[...] = o

    @pl.when(i == 0)
    def _():
        acc_ref[...] = jnp.zeros_like(acc_ref)

    acc_ref[0:1, :] += jnp.sum(o, axis=0, keepdims=True)
    acc_ref[1:2, :] += jnp.sum(o * o, axis=0, keepdims=True)

    @pl.when(i == pl.num_programs(0) - 1)
    def _():
        m = acc_ref[0:1, :] / M
        ex2 = acc_ref[1:2, :] / M
        st_ref[...] = jnp.concatenate([m, ex2 - m * m], axis=0)


def _final_kernel(o_ref, x2_ref, st_ref, g_ref, b_ref, out_ref):
    o = o_ref[...]
    on = (o - st_ref[0:1, :]) * jax.lax.rsqrt(st_ref[1:2, :] + EPS)
    out_ref[...] = x2_ref[...] + on * g_ref[...] + b_ref[...]


def kernel(x, g1, b1, Wqkv, Wo, g2, b2, W1, bb1, gf1, bf1, W2, bb2, gf2, bf2):
    T, B, C, Lt, Lh, Lw = x.shape
    nt, nh, nw = NWIN
    wt, wh, ww = Lt // nt, Lh // nh, Lw // nw
    P = nt * nh * nw
    w = wt * wh * ww
    N = B * P * w
    M = T * N
    Cf = 4 * C
    cb = 128
    ncb = C // cb

    xw = x.transpose(0, 1, 3, 4, 5, 2)
    xw = xw.reshape(T, B, nt, wt, nh, wh, nw, ww, C)
    xw = xw.transpose(0, 1, 2, 4, 6, 3, 5, 7, 8)
    Xw = xw.reshape(T, N, C)

    g1r = g1.reshape(1, 1, C)
    b1r = b1.reshape(1, 1, C)
    g2r = g2.reshape(1, 1, C)
    b2r = b2.reshape(1, 1, C)

    s1 = pl.pallas_call(
        functools.partial(_bnlif_kernel, nck=8),
        grid=(ncb,),
        in_specs=[
            pl.BlockSpec((T, N, cb), lambda i: (0, 0, i)),
            pl.BlockSpec((1, 1, cb), lambda i: (0, 0, i)),
            pl.BlockSpec((1, 1, cb), lambda i: (0, 0, i)),
        ],
        out_specs=pl.BlockSpec((T, N, cb), lambda i: (0, 0, i)),
        out_shape=jax.ShapeDtypeStruct((T, N, C), jnp.float32),
    )(Xw, g1r, b1r)

    TB = T * B
    Pw = P * w
    s1b = s1.reshape(T, B, Pw, C).reshape(TB, Pw, C)
    qkv, idx = pl.pallas_call(
        functools.partial(_route_kernel, P=P, w=w, C=C),
        grid=(TB,),
        in_specs=[
            pl.BlockSpec((1, Pw, C), lambda i: (i, 0, 0)),
            pl.BlockSpec((C, 3 * C), lambda i: (0, 0)),
        ],
        out_specs=[
            pl.BlockSpec((1, Pw, 3 * C), lambda i: (i, 0, 0)),
            pl.BlockSpec((1, K_TOP, P), lambda i: (i, 0, 0)),
        ],
        out_shape=[
            jax.ShapeDtypeStruct((TB, Pw, 3 * C), jnp.float32),
            jax.ShapeDtypeStruct((TB, K_TOP, P), jnp.int32),
        ],
    )(s1b, Wqkv)

    ao = pl.pallas_call(
        functools.partial(_attn_kernel, P=P, w=w, C=C),
        grid_spec=pltpu.PrefetchScalarGridSpec(
            num_scalar_prefetch=1,
            grid=(TB,),
            in_specs=[
                pl.BlockSpec((1, Pw, 3 * C), lambda b, idx_ref: (b, 0, 0)),
                pl.BlockSpec((C, C), lambda b, idx_ref: (0, 0)),
            ],
            out_specs=pl.BlockSpec((1, Pw, C), lambda b, idx_ref: (b, 0, 0)),
        ),
        out_shape=jax.ShapeDtypeStruct((TB, Pw, C), jnp.float32),
    )(idx, qkv, Wo)

    aow = ao.reshape(T, N, C)
    nb2 = 8
    x2 = pl.pallas_call(
        _add_kernel,
        grid=(nb2,),
        in_specs=[
            pl.BlockSpec((T, N // nb2, C), lambda i: (0, i, 0)),
            pl.BlockSpec((T, N // nb2, C), lambda i: (0, i, 0)),
        ],
        out_specs=pl.BlockSpec((T, N // nb2, C), lambda i: (0, i, 0)),
        out_shape=jax.ShapeDtypeStruct((T, N, C), jnp.float32),
    )(Xw, aow)
    s2 = pl.pallas_call(
        functools.partial(_bnlif_kernel, nck=8),
        grid=(ncb,),
        in_specs=[
            pl.BlockSpec((T, N, cb), lambda i: (0, 0, i)),
            pl.BlockSpec((1, 1, cb), lambda i: (0, 0, i)),
            pl.BlockSpec((1, 1, cb), lambda i: (0, 0, i)),
        ],
        out_specs=pl.BlockSpec((T, N, cb), lambda i: (0, 0, i)),
        out_shape=jax.ShapeDtypeStruct((T, N, C), jnp.float32),
    )(x2, g2r, b2r)

    s2v = s2.reshape(M, C)
    x2v = x2.reshape(M, C)
    mb = 512
    nmb = M // mb
    W1T = W1.T
    W2T = W2.T
    W1Tb = W1T.astype(jnp.bfloat16)
    W2Tb = W2T.astype(jnp.bfloat16)

    st1 = pl.pallas_call(
        functools.partial(_ffnstats_kernel, M=float(M)),
        grid=(nmb,),
        in_specs=[
            pl.BlockSpec((mb, C), lambda i: (i, 0)),
            pl.BlockSpec((C, Cf), lambda i: (0, 0)),
            pl.BlockSpec((1, Cf), lambda i: (0, 0)),
        ],
        out_specs=pl.BlockSpec((2, Cf), lambda i: (0, 0)),
        out_shape=jax.ShapeDtypeStruct((2, Cf), jnp.float32),
        scratch_shapes=[pltpu.VMEM((1, C), jnp.float32),
                        pltpu.VMEM((C, C), jnp.float32)],
    )(s2v, W1T, bb1.reshape(1, Cf))

    o2, st2 = pl.pallas_call(
        functools.partial(_ffn_kernel, M=float(M)),
        grid=(nmb,),
        in_specs=[
            pl.BlockSpec((mb, C), lambda i: (i, 0)),
            pl.BlockSpec((C, Cf), lambda i: (0, 0)),
            pl.BlockSpec((1, Cf), lambda i: (0, 0)),
            pl.BlockSpec((2, Cf), lambda i: (0, 0)),
            pl.BlockSpec((1, Cf), lambda i: (0, 0)),
            pl.BlockSpec((1, Cf), lambda i: (0, 0)),
            pl.BlockSpec((Cf, C), lambda i: (0, 0)),
            pl.BlockSpec((1, C), lambda i: (0, 0)),
        ],
        out_specs=[
            pl.BlockSpec((mb, C), lambda i: (i, 0)),
            pl.BlockSpec((2, C), lambda i: (0, 0)),
        ],
        out_shape=[
            jax.ShapeDtypeStruct((M, C), jnp.float32),
            jax.ShapeDtypeStruct((2, C), jnp.float32),
        ],
        scratch_shapes=[pltpu.VMEM((2, C), jnp.float32)],
    )(s2v, W1Tb, bb1.reshape(1, Cf), st1, gf1.reshape(1, Cf),
      bf1.reshape(1, Cf), W2Tb, bb2.reshape(1, C))

    outv = pl.pallas_call(
        _final_kernel,
        grid=(nmb,),
        in_specs=[
            pl.BlockSpec((mb, C), lambda i: (i, 0)),
            pl.BlockSpec((mb, C), lambda i: (i, 0)),
            pl.BlockSpec((2, C), lambda i: (0, 0)),
            pl.BlockSpec((1, C), lambda i: (0, 0)),
            pl.BlockSpec((1, C), lambda i: (0, 0)),
        ],
        out_specs=pl.BlockSpec((mb, C), lambda i: (i, 0)),
        out_shape=jax.ShapeDtypeStruct((M, C), jnp.float32),
    )(o2, x2v, st2, gf2.reshape(1, C), bf2.reshape(1, C))

    out = outv.reshape(T, B, nt, nh, nw, wt, wh, ww, C)
    out = out.transpose(0, 1, 2, 5, 3, 6, 4, 7, 8)
    out = out.reshape(T, B, Lt, Lh, Lw, C)
    return out.transpose(0, 1, 5, 2, 3, 4)


# speedup vs baseline: 1.1612x; 1.0015x over previous
import functools

import jax
import jax.numpy as jnp
from jax.experimental import pallas as pl
from jax.experimental.pallas import tpu as pltpu

TAU = 2.0
THR = 1.0
NWIN = (2, 4, 4)
K_TOP = 4
N_HEADS = 8
EPS = 1e-5


def _bnlif_kernel(x_ref, g_ref, b_ref, s_ref, *, nck):
    T, N, cb = x_ref.shape
    ck = N // nck
    acc = jnp.zeros((1, cb), jnp.float32)
    acc2 = jnp.zeros((1, cb), jnp.float32)
    for c in range(nck):
        xc = x_ref[:, c * ck:(c + 1) * ck, :]
        acc = acc + jnp.sum(xc, axis=(0, 1), keepdims=False)[None]
        acc2 = acc2 + jnp.sum(xc * xc, axis=(0, 1), keepdims=False)[None]
    cnt = float(T * N)
    m = acc / cnt
    var = acc2 / cnt - m * m
    scale = jax.lax.rsqrt(var + EPS) * g_ref[0]
    shift = b_ref[0] - m * scale
    for c in range(nck):
        sl = pl.ds(c * ck, ck)
        v = jnp.zeros((ck, cb), jnp.float32)
        for t in range(T):
            xn = x_ref[t, sl, :] * scale + shift
            v = v + (xn - v) / TAU
            sp = (v - THR >= 0).astype(jnp.float32)
            s_ref[t, sl, :] = sp
            v = v - sp * THR


def _add_kernel(x_ref, a_ref, o_ref):
    o_ref[...] = x_ref[...] + a_ref[...]


def _route_kernel(s_ref, wqkv_ref, q_out_ref, kv_out_ref, idx_ref, *, P, w, C):
    s = s_ref[0]
    qkv = jnp.dot(s, wqkv_ref[...], preferred_element_type=jnp.float32)
    q_out_ref[0] = qkv[:, :C]
    kv_out_ref[0] = qkv[:, C:]
    q = qkv[:, :C]
    k = qkv[:, C:2 * C]
    qm = jnp.mean(q.reshape(P, w, C), axis=1)
    km = jnp.mean(k.reshape(P, w, C), axis=1)
    aff = jax.lax.dot_general(qm, km, (((1,), (1,)), ((), ())),
                              preferred_element_type=jnp.float32)
    col = jax.lax.broadcasted_iota(jnp.int32, (P, P), 1)
    cur = aff
    for j in range(K_TOP):
        mx = jnp.max(cur, axis=1, keepdims=True)
        cand = jnp.where(cur == mx, col, jnp.int32(2 ** 30))
        idxj = jnp.min(cand, axis=1)
        idx_ref[0, j, :] = idxj
        cur = jnp.where(col == idxj[:, None], -jnp.inf, cur)


def _attn_kernel(idx_sref, q_ref, kv_ref, wo_ref, o_ref, kv_scr, *, w, C, G):
    b = pl.program_id(0)
    pg = pl.program_id(1)
    dh = C // N_HEADS
    scale = dh ** -0.5
    outs_w = []
    for u in range(G):
        p = pg * G + u
        for j in range(K_TOP):
            wi = idx_sref[b, j, p]
            kv_scr[u, pl.ds(j * w, w), :] = kv_ref[0, pl.ds(wi * w, w), :]
        houts = []
        for h in range(N_HEADS):
            qh = q_ref[0, u * w:(u + 1) * w, h * dh:(h + 1) * dh]
            kh = kv_scr[u, :, h * dh:(h + 1) * dh]
            vh = kv_scr[u, :, C + h * dh:C + (h + 1) * dh]
            logits = jax.lax.dot_general(
                qh, kh, (((1,), (1,)), ((), ())),
                preferred_element_type=jnp.float32) * scale
            logits = logits - jnp.max(logits, axis=1, keepdims=True)
            e = jnp.exp(logits)
            att = e / jnp.sum(e, axis=1, keepdims=True)
            houts.append(jnp.dot(att, vh, preferred_element_type=jnp.float32))
        outs_w.append(jnp.concatenate(houts, axis=1))
    o_all = jnp.concatenate(outs_w, axis=0)
    o_ref[0] = jnp.dot(o_all, wo_ref[...], preferred_element_type=jnp.float32)


def _ffnstats_kernel(s_ref, w1_ref, bb_ref, st_ref, cs_ref, gram_ref, *, M):
    i = pl.program_id(0)
    s = s_ref[...]

    @pl.when(i == 0)
    def _():
        cs_ref[...] = jnp.zeros_like(cs_ref)
        gram_ref[...] = jnp.zeros_like(gram_ref)

    cs_ref[...] += jnp.sum(s, axis=0, keepdims=True)
    sb = s.astype(jnp.bfloat16)
    gram_ref[...] += jax.lax.dot_general(
        sb, sb, (((0,), (0,)), ((), ())), preferred_element_type=jnp.float32)

    @pl.when(i == pl.num_programs(0) - 1)
    def _():
        wmat = w1_ref[...]
        sw = jnp.dot(gram_ref[...], wmat, preferred_element_type=jnp.float32)
        diag = jnp.sum(wmat * sw, axis=0, keepdims=True)
        cw = jnp.dot(cs_ref[...], wmat, preferred_element_type=jnp.float32)
        bb = bb_ref[...]
        sumh = cw + M * bb
        sumh2 = diag + 2.0 * bb * cw + M * bb * bb
        m = sumh / M
        var = sumh2 / M - m * m
        st_ref[...] = jnp.concatenate([m, var], axis=0)


def _ffn_kernel(s_ref, w1_ref, bb1_ref, st1_ref, g_ref, b_ref,
                w2_ref, bb2_ref, o_ref, st_ref, acc_ref, *, M):
    i = pl.program_id(0)
    sb = s_ref[...].astype(jnp.bfloat16)
    h = jnp.dot(sb, w1_ref[...],
                preferred_element_type=jnp.float32) + bb1_ref[...]
    hn = (h - st1_ref[0:1, :]) * jax.lax.rsqrt(st1_ref[1:2, :] + EPS)
    hn = hn * g_ref[...] + b_ref[...]
    hg = 0.5 * hn * (1.0 + jax.lax.erf(hn * (2.0 ** -0.5)))
    o = jnp.dot(hg.astype(jnp.bfloat16), w2_ref[...],
                preferred_element_type=jnp.float32) + bb2_ref[...]
    o_ref[...] = o

    @pl.when(i == 0)
    def _():
        acc_ref[...] = jnp.zeros_like(acc_ref)

    acc_ref[0:1, :] += jnp.sum(o, axis=0, keepdims=True)
    acc_ref[1:2, :] += jnp.sum(o * o, axis=0, keepdims=True)

    @pl.when(i == pl.num_programs(0) - 1)
    def _():
        m = acc_ref[0:1, :] / M
        ex2 = acc_ref[1:2, :] / M
        st_ref[...] = jnp.concatenate([m, ex2 - m * m], axis=0)


def _final_kernel(o_ref, x2_ref, st_ref, g_ref, b_ref, out_ref):
    o = o_ref[...]
    on = (o - st_ref[0:1, :]) * jax.lax.rsqrt(st_ref[1:2, :] + EPS)
    out_ref[...] = x2_ref[...] + on * g_ref[...] + b_ref[...]


def kernel(x, g1, b1, Wqkv, Wo, g2, b2, W1, bb1, gf1, bf1, W2, bb2, gf2, bf2):
    T, B, C, Lt, Lh, Lw = x.shape
    nt, nh, nw = NWIN
    wt, wh, ww = Lt // nt, Lh // nh, Lw // nw
    P = nt * nh * nw
    w = wt * wh * ww
    N = B * P * w
    M = T * N
    Cf = 4 * C
    cb = 128
    ncb = C // cb

    xw = x.transpose(0, 1, 3, 4, 5, 2)
    xw = xw.reshape(T, B, nt, wt, nh, wh, nw, ww, C)
    xw = xw.transpose(0, 1, 2, 4, 6, 3, 5, 7, 8)
    Xw = xw.reshape(T, N, C)

    g1r = g1.reshape(1, 1, C)
    b1r = b1.reshape(1, 1, C)
    g2r = g2.reshape(1, 1, C)
    b2r = b2.reshape(1, 1, C)

    s1 = pl.pallas_call(
        functools.partial(_bnlif_kernel, nck=8),
        grid=(ncb,),
        in_specs=[
            pl.BlockSpec((T, N, cb), lambda i: (0, 0, i)),
            pl.BlockSpec((1, 1, cb), lambda i: (0, 0, i)),
            pl.BlockSpec((1, 1, cb), lambda i: (0, 0, i)),
        ],
        out_specs=pl.BlockSpec((T, N, cb), lambda i: (0, 0, i)),
        out_shape=jax.ShapeDtypeStruct((T, N, C), jnp.float32),
    )(Xw, g1r, b1r)

    TB = T * B
    Pw = P * w
    s1b = s1.reshape(T, B, Pw, C).reshape(TB, Pw, C)
    q, kv, idx = pl.pallas_call(
        functools.partial(_route_kernel, P=P, w=w, C=C),
        grid=(TB,),
        in_specs=[
            pl.BlockSpec((1, Pw, C), lambda i: (i, 0, 0)),
            pl.BlockSpec((C, 3 * C), lambda i: (0, 0)),
        ],
        out_specs=[
            pl.BlockSpec((1, Pw, C), lambda i: (i, 0, 0)),
            pl.BlockSpec((1, Pw, 2 * C), lambda i: (i, 0, 0)),
            pl.BlockSpec((1, K_TOP, P), lambda i: (i, 0, 0)),
        ],
        out_shape=[
            jax.ShapeDtypeStruct((TB, Pw, C), jnp.float32),
            jax.ShapeDtypeStruct((TB, Pw, 2 * C), jnp.float32),
            jax.ShapeDtypeStruct((TB, K_TOP, P), jnp.int32),
        ],
    )(s1b, Wqkv)

    G = 4
    ao = pl.pallas_call(
        functools.partial(_attn_kernel, w=w, C=C, G=G),
        grid_spec=pltpu.PrefetchScalarGridSpec(
            num_scalar_prefetch=1,
            grid=(TB, P // G),
            in_specs=[
                pl.BlockSpec((1, G * w, C), lambda b, pg, idx_ref: (b, pg, 0)),
                pl.BlockSpec((1, Pw, 2 * C), lambda b, pg, idx_ref: (b, 0, 0)),
                pl.BlockSpec((C, C), lambda b, pg, idx_ref: (0, 0)),
            ],
            out_specs=pl.BlockSpec((1, G * w, C),
                                   lambda b, pg, idx_ref: (b, pg, 0)),
            scratch_shapes=[pltpu.VMEM((G, K_TOP * w, 2 * C), jnp.float32)],
        ),
        out_shape=jax.ShapeDtypeStruct((TB, Pw, C), jnp.float32),
    )(idx, q, kv, Wo)

    aow = ao.reshape(T, N, C)
    nb2 = 8
    x2 = pl.pallas_call(
        _add_kernel,
        grid=(nb2,),
        in_specs=[
            pl.BlockSpec((T, N // nb2, C), lambda i: (0, i, 0)),
            pl.BlockSpec((T, N // nb2, C), lambda i: (0, i, 0)),
        ],
        out_specs=pl.BlockSpec((T, N // nb2, C), lambda i: (0, i, 0)),
        out_shape=jax.ShapeDtypeStruct((T, N, C), jnp.float32),
    )(Xw, aow)
    s2 = pl.pallas_call(
        functools.partial(_bnlif_kernel, nck=8),
        grid=(ncb,),
        in_specs=[
            pl.BlockSpec((T, N, cb), lambda i: (0, 0, i)),
            pl.BlockSpec((1, 1, cb), lambda i: (0, 0, i)),
            pl.BlockSpec((1, 1, cb), lambda i: (0, 0, i)),
        ],
        out_specs=pl.BlockSpec((T, N, cb), lambda i: (0, 0, i)),
        out_shape=jax.ShapeDtypeStruct((T, N, C), jnp.float32),
    )(x2, g2r, b2r)

    s2v = s2.reshape(M, C)
    x2v = x2.reshape(M, C)
    mb = 512
    nmb = M // mb
    W1T = W1.T
    W2T = W2.T
    W1Tb = W1T.astype(jnp.bfloat16)
    W2Tb = W2T.astype(jnp.bfloat16)

    st1 = pl.pallas_call(
        functools.partial(_ffnstats_kernel, M=float(M)),
        grid=(nmb,),
        in_specs=[
            pl.BlockSpec((mb, C), lambda i: (i, 0)),
            pl.BlockSpec((C, Cf), lambda i: (0, 0)),
            pl.BlockSpec((1, Cf), lambda i: (0, 0)),
        ],
        out_specs=pl.BlockSpec((2, Cf), lambda i: (0, 0)),
        out_shape=jax.ShapeDtypeStruct((2, Cf), jnp.float32),
        scratch_shapes=[pltpu.VMEM((1, C), jnp.float32),
                        pltpu.VMEM((C, C), jnp.float32)],
    )(s2v, W1T, bb1.reshape(1, Cf))

    o2, st2 = pl.pallas_call(
        functools.partial(_ffn_kernel, M=float(M)),
        grid=(nmb,),
        in_specs=[
            pl.BlockSpec((mb, C), lambda i: (i, 0)),
            pl.BlockSpec((C, Cf), lambda i: (0, 0)),
            pl.BlockSpec((1, Cf), lambda i: (0, 0)),
            pl.BlockSpec((2, Cf), lambda i: (0, 0)),
            pl.BlockSpec((1, Cf), lambda i: (0, 0)),
            pl.BlockSpec((1, Cf), lambda i: (0, 0)),
            pl.BlockSpec((Cf, C), lambda i: (0, 0)),
            pl.BlockSpec((1, C), lambda i: (0, 0)),
        ],
        out_specs=[
            pl.BlockSpec((mb, C), lambda i: (i, 0)),
            pl.BlockSpec((2, C), lambda i: (0, 0)),
        ],
        out_shape=[
            jax.ShapeDtypeStruct((M, C), jnp.float32),
            jax.ShapeDtypeStruct((2, C), jnp.float32),
        ],
        scratch_shapes=[pltpu.VMEM((2, C), jnp.float32)],
    )(s2v, W1Tb, bb1.reshape(1, Cf), st1, gf1.reshape(1, Cf),
      bf1.reshape(1, Cf), W2Tb, bb2.reshape(1, C))

    outv = pl.pallas_call(
        _final_kernel,
        grid=(nmb,),
        in_specs=[
            pl.BlockSpec((mb, C), lambda i: (i, 0)),
            pl.BlockSpec((mb, C), lambda i: (i, 0)),
            pl.BlockSpec((2, C), lambda i: (0, 0)),
            pl.BlockSpec((1, C), lambda i: (0, 0)),
            pl.BlockSpec((1, C), lambda i: (0, 0)),
        ],
        out_specs=pl.BlockSpec((mb, C), lambda i: (i, 0)),
        out_shape=jax.ShapeDtypeStruct((M, C), jnp.float32),
    )(o2, x2v, st2, gf2.reshape(1, C), bf2.reshape(1, C))

    out = outv.reshape(T, B, nt, nh, nw, wt, wh, ww, C)
    out = out.transpose(0, 1, 2, 5, 3, 6, 4, 7, 8)
    out = out.reshape(T, B, Lt, Lh, Lw, C)
    return out.transpose(0, 1, 5, 2, 3, 4)


# speedup vs baseline: 2.8994x; 2.4970x over previous
import functools

import jax
import jax.numpy as jnp
from jax.experimental import pallas as pl
from jax.experimental.pallas import tpu as pltpu

TAU = 2.0
THR = 1.0
NWIN = (2, 4, 4)
K_TOP = 4
N_HEADS = 8
EPS = 1e-5


def _bnlif_kernel(x_ref, g_ref, b_ref, s_ref, *, nck):
    T, N, cb = x_ref.shape
    ck = N // nck
    acc = jnp.zeros((1, cb), jnp.float32)
    acc2 = jnp.zeros((1, cb), jnp.float32)
    for c in range(nck):
        xc = x_ref[:, c * ck:(c + 1) * ck, :]
        acc = acc + jnp.sum(xc, axis=(0, 1), keepdims=False)[None]
        acc2 = acc2 + jnp.sum(xc * xc, axis=(0, 1), keepdims=False)[None]
    cnt = float(T * N)
    m = acc / cnt
    var = acc2 / cnt - m * m
    scale = jax.lax.rsqrt(var + EPS) * g_ref[0]
    shift = b_ref[0] - m * scale
    for c in range(nck):
        sl = pl.ds(c * ck, ck)
        v = jnp.zeros((ck, cb), jnp.float32)
        for t in range(T):
            xn = x_ref[t, sl, :] * scale + shift
            v = v + (xn - v) / TAU
            sp = (v - THR >= 0).astype(jnp.float32)
            s_ref[t, sl, :] = sp
            v = v - sp * THR


def _add_kernel(x_ref, a_ref, o_ref):
    o_ref[...] = x_ref[...] + a_ref[...]


def _route_kernel(s_ref, wqkv_ref, q_out_ref, kt_ref, v_out_ref, idx_ref,
                  *, P, w, C):
    s = s_ref[0]
    qkv = jnp.dot(s, wqkv_ref[...], preferred_element_type=jnp.float32)
    q_out_ref[0] = qkv[:, :C]
    v_out_ref[0] = qkv[:, 2 * C:]
    q = qkv[:, :C]
    k = qkv[:, C:2 * C]
    kt_ref[0] = jnp.swapaxes(k.reshape(P, w, C), 1, 2)
    qm = jnp.mean(q.reshape(P, w, C), axis=1)
    km = jnp.mean(k.reshape(P, w, C), axis=1)
    aff = jax.lax.dot_general(qm, km, (((1,), (1,)), ((), ())),
                              preferred_element_type=jnp.float32)
    col = jax.lax.broadcasted_iota(jnp.int32, (P, P), 1)
    cur = aff
    for j in range(K_TOP):
        mx = jnp.max(cur, axis=1, keepdims=True)
        cand = jnp.where(cur == mx, col, jnp.int32(2 ** 30))
        idxj = jnp.min(cand, axis=1)
        idx_ref[0, j, :] = idxj
        cur = jnp.where(col == idxj[:, None], -jnp.inf, cur)


def _attn_kernel(idx_sref, q_ref, kt_ref, v_ref, wo_ref, xw_ref, x2_ref,
                 *scr, w, C, G):
    b = pl.program_id(0)
    pg = pl.program_id(1)
    H = N_HEADS
    dh = C // H
    scale = dh ** -0.5
    kt_scr = scr[:G]
    v_scr = scr[G:2 * G]
    qbd_scr = scr[2 * G:]
    lane = jax.lax.broadcasted_iota(jnp.int32, (1, C), 1)
    masks = [((lane >= h * dh) & (lane < (h + 1) * dh)).astype(jnp.bfloat16)
             for h in range(H)]
    for u in range(G):
        p = pg * G + u
        for j in range(K_TOP):
            wi = idx_sref[b, j, p]
            kt_scr[u][:, j * w:(j + 1) * w] = (
                kt_ref[0, wi, :, :].astype(jnp.bfloat16))
            v_scr[u][pl.ds(j * w, w), :] = (
                v_ref[0, pl.ds(wi * w, w), :].astype(jnp.bfloat16))
        qu = q_ref[0, u * w:(u + 1) * w, :].astype(jnp.bfloat16)
        for h in range(H):
            qbd_scr[u][h * w:(h + 1) * w, :] = qu * masks[h]
    outs_w = []
    for u in range(G):
        logits = jnp.dot(qbd_scr[u][...], kt_scr[u][...],
                         preferred_element_type=jnp.float32) * scale
        logits = logits - jnp.max(logits, axis=1, keepdims=True)
        e = jnp.exp(logits)
        att = e / jnp.sum(e, axis=1, keepdims=True)
        ov = jnp.dot(att.astype(jnp.bfloat16), v_scr[u][...],
                     preferred_element_type=jnp.float32)
        o_u = ov[0:w, :] * masks[0]
        for h in range(1, H):
            o_u = o_u + ov[h * w:(h + 1) * w, :] * masks[h]
        outs_w.append(o_u)
    o_all = jnp.concatenate(outs_w, axis=0)
    x2_ref[0] = xw_ref[0] + jnp.dot(o_all, wo_ref[...],
                                    preferred_element_type=jnp.float32)


def _bnlif_gram_kernel(x_ref, g_ref, b_ref, w1_ref, bb_ref, s_ref, st_ref,
                       acc_ref, cs_ref, gram_ref, *, M):
    ph = pl.program_id(0)
    i = pl.program_id(1)
    T, nb, C = x_ref.shape

    @pl.when(ph == 0)
    def _():
        @pl.when(i == 0)
        def _():
            acc_ref[...] = jnp.zeros_like(acc_ref)

        x = x_ref[...]
        acc_ref[0:1, :] += jnp.sum(x, axis=(0, 1), keepdims=False)[None]
        acc_ref[1:2, :] += jnp.sum(x * x, axis=(0, 1), keepdims=False)[None]

    @pl.when(ph == 1)
    def _():
        @pl.when(i == 0)
        def _():
            cs_ref[...] = jnp.zeros_like(cs_ref)
            gram_ref[...] = jnp.zeros_like(gram_ref)

        m = acc_ref[0:1, :] / M
        var = acc_ref[1:2, :] / M - m * m
        scale = jax.lax.rsqrt(var + EPS) * g_ref[...]
        shift = b_ref[...] - m * scale
        v = jnp.zeros((nb, C), jnp.float32)
        sts = []
        for t in range(T):
            xn = x_ref[t] * scale + shift
            v = v + (xn - v) / TAU
            sp = (v - THR >= 0).astype(jnp.float32)
            s_ref[t] = sp
            sts.append(sp)
            v = v - sp * THR
        sall = jnp.concatenate(sts, axis=0).astype(jnp.bfloat16)
        cs_ref[...] += jnp.sum(sall.astype(jnp.float32), axis=0,
                               keepdims=True)
        gram_ref[...] += jax.lax.dot_general(
            sall, sall, (((0,), (0,)), ((), ())),
            preferred_element_type=jnp.float32)

        @pl.when(i == pl.num_programs(1) - 1)
        def _():
            wmat = w1_ref[...]
            sw = jnp.dot(gram_ref[...], wmat,
                         preferred_element_type=jnp.float32)
            diag = jnp.sum(wmat * sw, axis=0, keepdims=True)
            cw = jnp.dot(cs_ref[...], wmat, preferred_element_type=jnp.float32)
            bb = bb_ref[...]
            mh = (cw + M * bb) / M
            varh = (diag + 2.0 * bb * cw + M * bb * bb) / M - mh * mh
            st_ref[...] = jnp.concatenate([mh, varh], axis=0)


def _ffn_kernel(s_ref, w1_ref, bb1_ref, st1_ref, g_ref, b_ref,
                w2_ref, bb2_ref, x2_ref, g2_ref, b2_ref, out_ref,
                acc_ref, o2_scr, *, M, mb):
    ph = pl.program_id(0)
    i = pl.program_id(1)

    @pl.when(ph == 0)
    def _():
        @pl.when(i == 0)
        def _():
            acc_ref[...] = jnp.zeros_like(acc_ref)

        sb = s_ref[...].astype(jnp.bfloat16)
        h = jnp.dot(sb, w1_ref[...],
                    preferred_element_type=jnp.float32) + bb1_ref[...]
        hn = (h - st1_ref[0:1, :]) * jax.lax.rsqrt(st1_ref[1:2, :] + EPS)
        hn = hn * g_ref[...] + b_ref[...]
        hg = 0.5 * hn * (1.0 + jax.lax.erf(hn * (2.0 ** -0.5)))
        o = jnp.dot(hg.astype(jnp.bfloat16), w2_ref[...],
                    preferred_element_type=jnp.float32) + bb2_ref[...]
        o2_scr[pl.ds(i * mb, mb), :] = o
        acc_ref[0:1, :] += jnp.sum(o, axis=0, keepdims=True)
        acc_ref[1:2, :] += jnp.sum(o * o, axis=0, keepdims=True)

    @pl.when(ph == 1)
    def _():
        m = acc_ref[0:1, :] / M
        var = acc_ref[1:2, :] / M - m * m
        o = o2_scr[pl.ds(i * mb, mb), :]
        on = (o - m) * jax.lax.rsqrt(var + EPS)
        out_ref[...] = x2_ref[...] + on * g2_ref[...] + b2_ref[...]


def kernel(x, g1, b1, Wqkv, Wo, g2, b2, W1, bb1, gf1, bf1, W2, bb2, gf2, bf2):
    T, B, C, Lt, Lh, Lw = x.shape
    nt, nh, nw = NWIN
    wt, wh, ww = Lt // nt, Lh // nh, Lw // nw
    P = nt * nh * nw
    w = wt * wh * ww
    N = B * P * w
    M = T * N
    Cf = 4 * C
    cb = 128
    ncb = C // cb

    xw = x.transpose(0, 1, 3, 4, 5, 2)
    xw = xw.reshape(T, B, nt, wt, nh, wh, nw, ww, C)
    xw = xw.transpose(0, 1, 2, 4, 6, 3, 5, 7, 8)
    Xw = xw.reshape(T, N, C)

    g1r = g1.reshape(1, 1, C)
    b1r = b1.reshape(1, 1, C)
    g2r = g2.reshape(1, 1, C)
    b2r = b2.reshape(1, 1, C)

    s1 = pl.pallas_call(
        functools.partial(_bnlif_kernel, nck=8),
        grid=(ncb,),
        in_specs=[
            pl.BlockSpec((T, N, cb), lambda i: (0, 0, i)),
            pl.BlockSpec((1, 1, cb), lambda i: (0, 0, i)),
            pl.BlockSpec((1, 1, cb), lambda i: (0, 0, i)),
        ],
        out_specs=pl.BlockSpec((T, N, cb), lambda i: (0, 0, i)),
        out_shape=jax.ShapeDtypeStruct((T, N, C), jnp.float32),
    )(Xw, g1r, b1r)

    TB = T * B
    Pw = P * w
    s1b = s1.reshape(T, B, Pw, C).reshape(TB, Pw, C)
    q, kt, v, idx = pl.pallas_call(
        functools.partial(_route_kernel, P=P, w=w, C=C),
        grid=(TB,),
        in_specs=[
            pl.BlockSpec((1, Pw, C), lambda i: (i, 0, 0)),
            pl.BlockSpec((C, 3 * C), lambda i: (0, 0)),
        ],
        out_specs=[
            pl.BlockSpec((1, Pw, C), lambda i: (i, 0, 0)),
            pl.BlockSpec((1, P, C, w), lambda i: (i, 0, 0, 0)),
            pl.BlockSpec((1, Pw, C), lambda i: (i, 0, 0)),
            pl.BlockSpec((1, K_TOP, P), lambda i: (i, 0, 0)),
        ],
        out_shape=[
            jax.ShapeDtypeStruct((TB, Pw, C), jnp.float32),
            jax.ShapeDtypeStruct((TB, P, C, w), jnp.float32),
            jax.ShapeDtypeStruct((TB, Pw, C), jnp.float32),
            jax.ShapeDtypeStruct((TB, K_TOP, P), jnp.int32),
        ],
    )(s1b, Wqkv)

    G = 8
    scr = ([pltpu.VMEM((C, K_TOP * w), jnp.bfloat16) for _ in range(G)]
           + [pltpu.VMEM((K_TOP * w, C), jnp.bfloat16) for _ in range(G)]
           + [pltpu.VMEM((N_HEADS * w, C), jnp.bfloat16) for _ in range(G)])
    Xwb = Xw.reshape(T, B, Pw, C).reshape(TB, Pw, C)
    x2b = pl.pallas_call(
        functools.partial(_attn_kernel, w=w, C=C, G=G),
        grid_spec=pltpu.PrefetchScalarGridSpec(
            num_scalar_prefetch=1,
            grid=(TB, P // G),
            in_specs=[
                pl.BlockSpec((1, G * w, C), lambda b, pg, idx_ref: (b, pg, 0)),
                pl.BlockSpec((1, P, C, w),
                             lambda b, pg, idx_ref: (b, 0, 0, 0)),
                pl.BlockSpec((1, Pw, C), lambda b, pg, idx_ref: (b, 0, 0)),
                pl.BlockSpec((C, C), lambda b, pg, idx_ref: (0, 0)),
                pl.BlockSpec((1, G * w, C), lambda b, pg, idx_ref: (b, pg, 0)),
            ],
            out_specs=pl.BlockSpec((1, G * w, C),
                                   lambda b, pg, idx_ref: (b, pg, 0)),
            scratch_shapes=scr,
        ),
        out_shape=jax.ShapeDtypeStruct((TB, Pw, C), jnp.float32),
    )(idx, q, kt, v, Wo, Xwb)

    x2 = x2b.reshape(T, N, C)
    W1T = W1.T
    W2Tb = W2.T.astype(jnp.bfloat16)
    W1Tb = W1T.astype(jnp.bfloat16)
    nb = 1024
    nnb = N // nb
    s2, st1 = pl.pallas_call(
        functools.partial(_bnlif_gram_kernel, M=float(M)),
        grid=(2, nnb),
        in_specs=[
            pl.BlockSpec((T, nb, C), lambda ph, i: (0, i, 0)),
            pl.BlockSpec((1, C), lambda ph, i: (0, 0)),
            pl.BlockSpec((1, C), lambda ph, i: (0, 0)),
            pl.BlockSpec((C, Cf), lambda ph, i: (0, 0)),
            pl.BlockSpec((1, Cf), lambda ph, i: (0, 0)),
        ],
        out_specs=[
            pl.BlockSpec((T, nb, C), lambda ph, i: (0, i * ph, 0)),
            pl.BlockSpec((2, Cf), lambda ph, i: (0, 0)),
        ],
        out_shape=[
            jax.ShapeDtypeStruct((T, N, C), jnp.float32),
            jax.ShapeDtypeStruct((2, Cf), jnp.float32),
        ],
        scratch_shapes=[pltpu.VMEM((2, C), jnp.float32),
                        pltpu.VMEM((1, C), jnp.float32),
                        pltpu.VMEM((C, C), jnp.float32)],
    )(x2, g2.reshape(1, C), b2.reshape(1, C), W1T, bb1.reshape(1, Cf))

    s2v = s2.reshape(M, C)
    x2v = x2.reshape(M, C)
    mb = 1024
    nmb = M // mb

    outv = pl.pallas_call(
        functools.partial(_ffn_kernel, M=float(M), mb=mb),
        grid=(2, nmb),
        in_specs=[
            pl.BlockSpec((mb, C), lambda ph, i: (i * (1 - ph), 0)),
            pl.BlockSpec((C, Cf), lambda ph, i: (0, 0)),
            pl.BlockSpec((1, Cf), lambda ph, i: (0, 0)),
            pl.BlockSpec((2, Cf), lambda ph, i: (0, 0)),
            pl.BlockSpec((1, Cf), lambda ph, i: (0, 0)),
            pl.BlockSpec((1, Cf), lambda ph, i: (0, 0)),
            pl.BlockSpec((Cf, C), lambda ph, i: (0, 0)),
            pl.BlockSpec((1, C), lambda ph, i: (0, 0)),
            pl.BlockSpec((mb, C), lambda ph, i: (i * ph, 0)),
            pl.BlockSpec((1, C), lambda ph, i: (0, 0)),
            pl.BlockSpec((1, C), lambda ph, i: (0, 0)),
        ],
        out_specs=pl.BlockSpec((mb, C), lambda ph, i: (i * ph, 0)),
        out_shape=jax.ShapeDtypeStruct((M, C), jnp.float32),
        scratch_shapes=[pltpu.VMEM((2, C), jnp.float32),
                        pltpu.VMEM((M, C), jnp.float32)],
    )(s2v, W1Tb, bb1.reshape(1, Cf), st1, gf1.reshape(1, Cf),
      bf1.reshape(1, Cf), W2Tb, bb2.reshape(1, C), x2v,
      gf2.reshape(1, C), bf2.reshape(1, C))

    out = outv.reshape(T, B, nt, nh, nw, wt, wh, ww, C)
    out = out.transpose(0, 1, 2, 5, 3, 6, 4, 7, 8)
    out = out.reshape(T, B, Lt, Lh, Lw, C)
    return out.transpose(0, 1, 5, 2, 3, 4)


# speedup vs baseline: 2.9401x; 1.0140x over previous
import functools

import jax
import jax.numpy as jnp
from jax.experimental import pallas as pl
from jax.experimental.pallas import tpu as pltpu

TAU = 2.0
THR = 1.0
NWIN = (2, 4, 4)
K_TOP = 4
N_HEADS = 8
EPS = 1e-5


def _bnlif_kernel(x_ref, g_ref, b_ref, s_ref, *, nck):
    T, N, cb = x_ref.shape
    ck = N // nck
    acc = jnp.zeros((1, cb), jnp.float32)
    acc2 = jnp.zeros((1, cb), jnp.float32)
    for c in range(nck):
        xc = x_ref[:, c * ck:(c + 1) * ck, :]
        acc = acc + jnp.sum(xc, axis=(0, 1), keepdims=False)[None]
        acc2 = acc2 + jnp.sum(xc * xc, axis=(0, 1), keepdims=False)[None]
    cnt = float(T * N)
    m = acc / cnt
    var = acc2 / cnt - m * m
    scale = jax.lax.rsqrt(var + EPS) * g_ref[0]
    shift = b_ref[0] - m * scale
    for c in range(nck):
        sl = pl.ds(c * ck, ck)
        v = jnp.zeros((ck, cb), jnp.float32)
        for t in range(T):
            xn = x_ref[t, sl, :] * scale + shift
            v = v + (xn - v) / TAU
            sp = (v - THR >= 0).astype(jnp.float32)
            s_ref[t, sl, :] = sp
            v = v - sp * THR


def _add_kernel(x_ref, a_ref, o_ref):
    o_ref[...] = x_ref[...] + a_ref[...]


def _route_kernel(s_ref, wqkv_ref, q_out_ref, kt_ref, v_out_ref, idx_ref,
                  *, P, w, C):
    s = s_ref[0]
    qkv = jnp.dot(s, wqkv_ref[...], preferred_element_type=jnp.float32)
    q = qkv[:, :C]
    k = qkv[:, C:2 * C]
    q_out_ref[0] = q.astype(jnp.bfloat16)
    v_out_ref[0] = qkv[:, 2 * C:].astype(jnp.bfloat16)
    kt_ref[0] = jnp.swapaxes(k.reshape(P, w, C), 1, 2).astype(jnp.bfloat16)
    qm = jnp.mean(q.reshape(P, w, C), axis=1)
    km = jnp.mean(k.reshape(P, w, C), axis=1)
    aff = jax.lax.dot_general(qm, km, (((1,), (1,)), ((), ())),
                              preferred_element_type=jnp.float32)
    col = jax.lax.broadcasted_iota(jnp.int32, (P, P), 1)
    cur = aff
    for j in range(K_TOP):
        mx = jnp.max(cur, axis=1, keepdims=True)
        cand = jnp.where(cur == mx, col, jnp.int32(2 ** 30))
        idxj = jnp.min(cand, axis=1)
        idx_ref[0, j, :] = idxj
        cur = jnp.where(col == idxj[:, None], -jnp.inf, cur)


def _attn_kernel(idx_sref, q_ref, kt_ref, v_ref, wo_ref, xw_ref, x2_ref,
                 *scr, w, C, G):
    b = pl.program_id(0)
    pg = pl.program_id(1)
    H = N_HEADS
    dh = C // H
    scale = dh ** -0.5
    kt_scr = scr[:G]
    v_scr = scr[G:2 * G]
    qbd_scr = scr[2 * G:]
    lane = jax.lax.broadcasted_iota(jnp.int32, (1, C), 1)
    masks = [((lane >= h * dh) & (lane < (h + 1) * dh)).astype(jnp.bfloat16)
             for h in range(H)]
    for u in range(G):
        p = pg * G + u
        for j in range(K_TOP):
            wi = idx_sref[b, j, p]
            kt_scr[u][:, j * w:(j + 1) * w] = kt_ref[0, wi, :, :]
            v_scr[u][pl.ds(j * w, w), :] = v_ref[0, pl.ds(wi * w, w), :]
        qu = q_ref[0, u * w:(u + 1) * w, :]
        for h in range(H):
            qbd_scr[u][h * w:(h + 1) * w, :] = qu * masks[h]
    outs_w = []
    for u in range(G):
        logits = jnp.dot(qbd_scr[u][...], kt_scr[u][...],
                         preferred_element_type=jnp.float32) * scale
        logits = logits - jnp.max(logits, axis=1, keepdims=True)
        e = jnp.exp(logits)
        att = e / jnp.sum(e, axis=1, keepdims=True)
        ov = jnp.dot(att.astype(jnp.bfloat16), v_scr[u][...],
                     preferred_element_type=jnp.float32)
        o_u = ov[0:w, :] * masks[0]
        for h in range(1, H):
            o_u = o_u + ov[h * w:(h + 1) * w, :] * masks[h]
        outs_w.append(o_u)
    o_all = jnp.concatenate(outs_w, axis=0)
    x2_ref[0] = xw_ref[0] + jnp.dot(o_all, wo_ref[...],
                                    preferred_element_type=jnp.float32)


def _bnlif_gram_kernel(x_ref, g_ref, b_ref, w1_ref, bb_ref, s_ref, st_ref,
                       acc_ref, cs_ref, gram_ref, *, M):
    ph = pl.program_id(0)
    i = pl.program_id(1)
    T, nb, C = x_ref.shape

    @pl.when(ph == 0)
    def _():
        @pl.when(i == 0)
        def _():
            acc_ref[...] = jnp.zeros_like(acc_ref)

        x = x_ref[...]
        acc_ref[0:1, :] += jnp.sum(x, axis=(0, 1), keepdims=False)[None]
        acc_ref[1:2, :] += jnp.sum(x * x, axis=(0, 1), keepdims=False)[None]

    @pl.when(ph == 1)
    def _():
        @pl.when(i == 0)
        def _():
            cs_ref[...] = jnp.zeros_like(cs_ref)
            gram_ref[...] = jnp.zeros_like(gram_ref)

        m = acc_ref[0:1, :] / M
        var = acc_ref[1:2, :] / M - m * m
        scale = jax.lax.rsqrt(var + EPS) * g_ref[...]
        shift = b_ref[...] - m * scale
        v = jnp.zeros((nb, C), jnp.float32)
        sts = []
        for t in range(T):
            xn = x_ref[t] * scale + shift
            v = v + (xn - v) / TAU
            sp = (v - THR >= 0).astype(jnp.float32)
            s_ref[t] = sp
            sts.append(sp)
            v = v - sp * THR
        sall = jnp.concatenate(sts, axis=0).astype(jnp.bfloat16)
        cs_ref[...] += jnp.sum(sall.astype(jnp.float32), axis=0,
                               keepdims=True)
        gram_ref[...] += jax.lax.dot_general(
            sall, sall, (((0,), (0,)), ((), ())),
            preferred_element_type=jnp.float32)

        @pl.when(i == pl.num_programs(1) - 1)
        def _():
            wmat = w1_ref[...]
            sw = jnp.dot(gram_ref[...], wmat,
                         preferred_element_type=jnp.float32)
            diag = jnp.sum(wmat * sw, axis=0, keepdims=True)
            cw = jnp.dot(cs_ref[...], wmat, preferred_element_type=jnp.float32)
            bb = bb_ref[...]
            mh = (cw + M * bb) / M
            varh = (diag + 2.0 * bb * cw + M * bb * bb) / M - mh * mh
            st_ref[...] = jnp.concatenate([mh, varh], axis=0)


def _ffn_kernel(s_ref, w1_ref, bb1_ref, st1_ref, g_ref, b_ref,
                w2_ref, bb2_ref, x2_ref, g2_ref, b2_ref, out_ref,
                acc_ref, o2_scr, *, M, mb):
    ph = pl.program_id(0)
    i = pl.program_id(1)

    @pl.when(ph == 0)
    def _():
        @pl.when(i == 0)
        def _():
            acc_ref[...] = jnp.zeros_like(acc_ref)

        sb = s_ref[...].astype(jnp.bfloat16)
        h = jnp.dot(sb, w1_ref[...],
                    preferred_element_type=jnp.float32) + bb1_ref[...]
        hn = (h - st1_ref[0:1, :]) * jax.lax.rsqrt(st1_ref[1:2, :] + EPS)
        hn = hn * g_ref[...] + b_ref[...]
        hg = 0.5 * hn * (1.0 + jax.lax.erf(hn * (2.0 ** -0.5)))
        o = jnp.dot(hg.astype(jnp.bfloat16), w2_ref[...],
                    preferred_element_type=jnp.float32) + bb2_ref[...]
        o2_scr[pl.ds(i * mb, mb), :] = o
        acc_ref[0:1, :] += jnp.sum(o, axis=0, keepdims=True)
        acc_ref[1:2, :] += jnp.sum(o * o, axis=0, keepdims=True)

    @pl.when(ph == 1)
    def _():
        m = acc_ref[0:1, :] / M
        var = acc_ref[1:2, :] / M - m * m
        o = o2_scr[pl.ds(i * mb, mb), :]
        on = (o - m) * jax.lax.rsqrt(var + EPS)
        out_ref[...] = x2_ref[...] + on * g2_ref[...] + b2_ref[...]


def kernel(x, g1, b1, Wqkv, Wo, g2, b2, W1, bb1, gf1, bf1, W2, bb2, gf2, bf2):
    T, B, C, Lt, Lh, Lw = x.shape
    nt, nh, nw = NWIN
    wt, wh, ww = Lt // nt, Lh // nh, Lw // nw
    P = nt * nh * nw
    w = wt * wh * ww
    N = B * P * w
    M = T * N
    Cf = 4 * C
    cb = 128
    ncb = C // cb

    xw = x.transpose(0, 1, 3, 4, 5, 2)
    xw = xw.reshape(T, B, nt, wt, nh, wh, nw, ww, C)
    xw = xw.transpose(0, 1, 2, 4, 6, 3, 5, 7, 8)
    Xw = xw.reshape(T, N, C)

    g1r = g1.reshape(1, 1, C)
    b1r = b1.reshape(1, 1, C)
    g2r = g2.reshape(1, 1, C)
    b2r = b2.reshape(1, 1, C)

    s1 = pl.pallas_call(
        functools.partial(_bnlif_kernel, nck=8),
        grid=(ncb,),
        in_specs=[
            pl.BlockSpec((T, N, cb), lambda i: (0, 0, i)),
            pl.BlockSpec((1, 1, cb), lambda i: (0, 0, i)),
            pl.BlockSpec((1, 1, cb), lambda i: (0, 0, i)),
        ],
        out_specs=pl.BlockSpec((T, N, cb), lambda i: (0, 0, i)),
        out_shape=jax.ShapeDtypeStruct((T, N, C), jnp.float32),
    )(Xw, g1r, b1r)

    TB = T * B
    Pw = P * w
    s1b = s1.reshape(T, B, Pw, C).reshape(TB, Pw, C)
    q, kt, v, idx = pl.pallas_call(
        functools.partial(_route_kernel, P=P, w=w, C=C),
        grid=(TB,),
        in_specs=[
            pl.BlockSpec((1, Pw, C), lambda i: (i, 0, 0)),
            pl.BlockSpec((C, 3 * C), lambda i: (0, 0)),
        ],
        out_specs=[
            pl.BlockSpec((1, Pw, C), lambda i: (i, 0, 0)),
            pl.BlockSpec((1, P, C, w), lambda i: (i, 0, 0, 0)),
            pl.BlockSpec((1, Pw, C), lambda i: (i, 0, 0)),
            pl.BlockSpec((1, K_TOP, P), lambda i: (i, 0, 0)),
        ],
        out_shape=[
            jax.ShapeDtypeStruct((TB, Pw, C), jnp.bfloat16),
            jax.ShapeDtypeStruct((TB, P, C, w), jnp.bfloat16),
            jax.ShapeDtypeStruct((TB, Pw, C), jnp.bfloat16),
            jax.ShapeDtypeStruct((TB, K_TOP, P), jnp.int32),
        ],
    )(s1b, Wqkv)

    G = 8
    scr = ([pltpu.VMEM((C, K_TOP * w), jnp.bfloat16) for _ in range(G)]
           + [pltpu.VMEM((K_TOP * w, C), jnp.bfloat16) for _ in range(G)]
           + [pltpu.VMEM((N_HEADS * w, C), jnp.bfloat16) for _ in range(G)])
    Xwb = Xw.reshape(T, B, Pw, C).reshape(TB, Pw, C)
    x2b = pl.pallas_call(
        functools.partial(_attn_kernel, w=w, C=C, G=G),
        grid_spec=pltpu.PrefetchScalarGridSpec(
            num_scalar_prefetch=1,
            grid=(TB, P // G),
            in_specs=[
                pl.BlockSpec((1, G * w, C), lambda b, pg, idx_ref: (b, pg, 0)),
                pl.BlockSpec((1, P, C, w),
                             lambda b, pg, idx_ref: (b, 0, 0, 0)),
                pl.BlockSpec((1, Pw, C), lambda b, pg, idx_ref: (b, 0, 0)),
                pl.BlockSpec((C, C), lambda b, pg, idx_ref: (0, 0)),
                pl.BlockSpec((1, G * w, C), lambda b, pg, idx_ref: (b, pg, 0)),
            ],
            out_specs=pl.BlockSpec((1, G * w, C),
                                   lambda b, pg, idx_ref: (b, pg, 0)),
            scratch_shapes=scr,
        ),
        out_shape=jax.ShapeDtypeStruct((TB, Pw, C), jnp.float32),
    )(idx, q, kt, v, Wo, Xwb)

    x2 = x2b.reshape(T, N, C)
    W1T = W1.T
    W2Tb = W2.T.astype(jnp.bfloat16)
    W1Tb = W1T.astype(jnp.bfloat16)
    nb = 1024
    nnb = N // nb
    s2, st1 = pl.pallas_call(
        functools.partial(_bnlif_gram_kernel, M=float(M)),
        grid=(2, nnb),
        in_specs=[
            pl.BlockSpec((T, nb, C), lambda ph, i: (0, i, 0)),
            pl.BlockSpec((1, C), lambda ph, i: (0, 0)),
            pl.BlockSpec((1, C), lambda ph, i: (0, 0)),
            pl.BlockSpec((C, Cf), lambda ph, i: (0, 0)),
            pl.BlockSpec((1, Cf), lambda ph, i: (0, 0)),
        ],
        out_specs=[
            pl.BlockSpec((T, nb, C), lambda ph, i: (0, i * ph, 0)),
            pl.BlockSpec((2, Cf), lambda ph, i: (0, 0)),
        ],
        out_shape=[
            jax.ShapeDtypeStruct((T, N, C), jnp.float32),
            jax.ShapeDtypeStruct((2, Cf), jnp.float32),
        ],
        scratch_shapes=[pltpu.VMEM((2, C), jnp.float32),
                        pltpu.VMEM((1, C), jnp.float32),
                        pltpu.VMEM((C, C), jnp.float32)],
    )(x2, g2.reshape(1, C), b2.reshape(1, C), W1T, bb1.reshape(1, Cf))

    s2v = s2.reshape(M, C)
    x2v = x2.reshape(M, C)
    mb = 1024
    nmb = M // mb

    outv = pl.pallas_call(
        functools.partial(_ffn_kernel, M=float(M), mb=mb),
        grid=(2, nmb),
        in_specs=[
            pl.BlockSpec((mb, C), lambda ph, i: (i * (1 - ph), 0)),
            pl.BlockSpec((C, Cf), lambda ph, i: (0, 0)),
            pl.BlockSpec((1, Cf), lambda ph, i: (0, 0)),
            pl.BlockSpec((2, Cf), lambda ph, i: (0, 0)),
            pl.BlockSpec((1, Cf), lambda ph, i: (0, 0)),
            pl.BlockSpec((1, Cf), lambda ph, i: (0, 0)),
            pl.BlockSpec((Cf, C), lambda ph, i: (0, 0)),
            pl.BlockSpec((1, C), lambda ph, i: (0, 0)),
            pl.BlockSpec((mb, C), lambda ph, i: (i * ph, 0)),
            pl.BlockSpec((1, C), lambda ph, i: (0, 0)),
            pl.BlockSpec((1, C), lambda ph, i: (0, 0)),
        ],
        out_specs=pl.BlockSpec((mb, C), lambda ph, i: (i * ph, 0)),
        out_shape=jax.ShapeDtypeStruct((M, C), jnp.float32),
        scratch_shapes=[pltpu.VMEM((2, C), jnp.float32),
                        pltpu.VMEM((M, C), jnp.float32)],
    )(s2v, W1Tb, bb1.reshape(1, Cf), st1, gf1.reshape(1, Cf),
      bf1.reshape(1, Cf), W2Tb, bb2.reshape(1, C), x2v,
      gf2.reshape(1, C), bf2.reshape(1, C))

    out = outv.reshape(T, B, nt, nh, nw, wt, wh, ww, C)
    out = out.transpose(0, 1, 2, 5, 3, 6, 4, 7, 8)
    out = out.reshape(T, B, Lt, Lh, Lw, C)
    return out.transpose(0, 1, 5, 2, 3, 4)


# speedup vs baseline: 2.9750x; 1.0118x over previous
import functools

import jax
import jax.numpy as jnp
from jax.experimental import pallas as pl
from jax.experimental.pallas import tpu as pltpu

TAU = 2.0
THR = 1.0
NWIN = (2, 4, 4)
K_TOP = 4
N_HEADS = 8
EPS = 1e-5


def _bnlif_kernel(x_ref, g_ref, b_ref, s_ref, *, nck):
    T, N, cb = x_ref.shape
    ck = N // nck
    acc = jnp.zeros((1, cb), jnp.float32)
    acc2 = jnp.zeros((1, cb), jnp.float32)
    for c in range(nck):
        xc = x_ref[:, c * ck:(c + 1) * ck, :]
        acc = acc + jnp.sum(xc, axis=(0, 1), keepdims=False)[None]
        acc2 = acc2 + jnp.sum(xc * xc, axis=(0, 1), keepdims=False)[None]
    cnt = float(T * N)
    m = acc / cnt
    var = acc2 / cnt - m * m
    scale = jax.lax.rsqrt(var + EPS) * g_ref[0]
    shift = b_ref[0] - m * scale
    for c in range(nck):
        sl = pl.ds(c * ck, ck)
        v = jnp.zeros((ck, cb), jnp.float32)
        for t in range(T):
            xn = x_ref[t, sl, :] * scale + shift
            v = v + (xn - v) / TAU
            sp = (v - THR >= 0).astype(jnp.float32)
            s_ref[t, sl, :] = sp.astype(jnp.bfloat16)
            v = v - sp * THR


def _add_kernel(x_ref, a_ref, o_ref):
    o_ref[...] = x_ref[...] + a_ref[...]


def _route_kernel(s_ref, wqkv_ref, q_out_ref, kt_ref, v_out_ref, idx_ref,
                  *, P, w, C):
    s = s_ref[0].astype(jnp.float32)
    qkv = jnp.dot(s, wqkv_ref[...], preferred_element_type=jnp.float32)
    q = qkv[:, :C]
    k = qkv[:, C:2 * C]
    q_out_ref[0] = q.astype(jnp.bfloat16)
    v_out_ref[0] = qkv[:, 2 * C:].astype(jnp.bfloat16)
    kt_ref[0] = jnp.swapaxes(k.reshape(P, w, C), 1, 2).astype(jnp.bfloat16)
    qm = jnp.mean(q.reshape(P, w, C), axis=1)
    km = jnp.mean(k.reshape(P, w, C), axis=1)
    aff = jax.lax.dot_general(qm, km, (((1,), (1,)), ((), ())),
                              preferred_element_type=jnp.float32)
    col = jax.lax.broadcasted_iota(jnp.int32, (P, P), 1)
    cur = aff
    for j in range(K_TOP):
        mx = jnp.max(cur, axis=1, keepdims=True)
        cand = jnp.where(cur == mx, col, jnp.int32(2 ** 30))
        idxj = jnp.min(cand, axis=1)
        idx_ref[0, j, :] = idxj
        cur = jnp.where(col == idxj[:, None], -jnp.inf, cur)


def _attn_kernel(idx_sref, q_ref, kt_ref, v_ref, wo_ref, xw_ref, x2_ref,
                 *scr, w, C, G):
    b = pl.program_id(0)
    pg = pl.program_id(1)
    H = N_HEADS
    dh = C // H
    scale = dh ** -0.5
    kt_scr = scr[:G]
    v_scr = scr[G:2 * G]
    qbd_scr = scr[2 * G:]
    lane = jax.lax.broadcasted_iota(jnp.int32, (1, C), 1)
    masks = [((lane >= h * dh) & (lane < (h + 1) * dh)).astype(jnp.bfloat16)
             for h in range(H)]
    for u in range(G):
        p = pg * G + u
        for j in range(K_TOP):
            wi = idx_sref[b, j, p]
            kt_scr[u][:, j * w:(j + 1) * w] = kt_ref[0, wi, :, :]
            v_scr[u][pl.ds(j * w, w), :] = v_ref[0, pl.ds(wi * w, w), :]
        qu = q_ref[0, u * w:(u + 1) * w, :]
        for h in range(H):
            qbd_scr[u][h * w:(h + 1) * w, :] = qu * masks[h]
    outs_w = []
    for u in range(G):
        logits = jnp.dot(qbd_scr[u][...], kt_scr[u][...],
                         preferred_element_type=jnp.float32) * scale
        logits = logits - jnp.max(logits, axis=1, keepdims=True)
        e = jnp.exp(logits)
        att = e / jnp.sum(e, axis=1, keepdims=True)
        ov = jnp.dot(att.astype(jnp.bfloat16), v_scr[u][...],
                     preferred_element_type=jnp.float32)
        o_u = ov[0:w, :] * masks[0]
        for h in range(1, H):
            o_u = o_u + ov[h * w:(h + 1) * w, :] * masks[h]
        outs_w.append(o_u)
    o_all = jnp.concatenate(outs_w, axis=0)
    x2_ref[0] = xw_ref[0] + jnp.dot(o_all, wo_ref[...],
                                    preferred_element_type=jnp.float32)


def _bnlif_gram_kernel(x_ref, g_ref, b_ref, w1_ref, bb_ref, s_ref, st_ref,
                       acc_ref, cs_ref, gram_ref, *, M):
    ph = pl.program_id(0)
    i = pl.program_id(1)
    T, nb, C = x_ref.shape

    @pl.when(ph == 0)
    def _():
        @pl.when(i == 0)
        def _():
            acc_ref[...] = jnp.zeros_like(acc_ref)

        x = x_ref[...]
        acc_ref[0:1, :] += jnp.sum(x, axis=(0, 1), keepdims=False)[None]
        acc_ref[1:2, :] += jnp.sum(x * x, axis=(0, 1), keepdims=False)[None]

    @pl.when(ph == 1)
    def _():
        @pl.when(i == 0)
        def _():
            cs_ref[...] = jnp.zeros_like(cs_ref)
            gram_ref[...] = jnp.zeros_like(gram_ref)

        m = acc_ref[0:1, :] / M
        var = acc_ref[1:2, :] / M - m * m
        scale = jax.lax.rsqrt(var + EPS) * g_ref[...]
        shift = b_ref[...] - m * scale
        v = jnp.zeros((nb, C), jnp.float32)
        sts = []
        for t in range(T):
            xn = x_ref[t] * scale + shift
            v = v + (xn - v) / TAU
            sp = (v - THR >= 0).astype(jnp.float32)
            s_ref[t] = sp.astype(jnp.bfloat16)
            sts.append(sp)
            v = v - sp * THR
        sall = jnp.concatenate(sts, axis=0).astype(jnp.bfloat16)
        cs_ref[...] += jnp.sum(sall.astype(jnp.float32), axis=0,
                               keepdims=True)
        gram_ref[...] += jax.lax.dot_general(
            sall, sall, (((0,), (0,)), ((), ())),
            preferred_element_type=jnp.float32)

        @pl.when(i == pl.num_programs(1) - 1)
        def _():
            wmat = w1_ref[...]
            sw = jnp.dot(gram_ref[...], wmat,
                         preferred_element_type=jnp.float32)
            diag = jnp.sum(wmat * sw, axis=0, keepdims=True)
            cw = jnp.dot(cs_ref[...], wmat, preferred_element_type=jnp.float32)
            bb = bb_ref[...]
            mh = (cw + M * bb) / M
            varh = (diag + 2.0 * bb * cw + M * bb * bb) / M - mh * mh
            st_ref[...] = jnp.concatenate([mh, varh], axis=0)


def _ffn_kernel(s_ref, w1_ref, bb1_ref, st1_ref, g_ref, b_ref,
                w2_ref, bb2_ref, x2_ref, g2_ref, b2_ref, out_ref,
                acc_ref, o2_scr, *, M, mb):
    ph = pl.program_id(0)
    i = pl.program_id(1)

    @pl.when(ph == 0)
    def _():
        @pl.when(i == 0)
        def _():
            acc_ref[...] = jnp.zeros_like(acc_ref)

        sb = s_ref[...]
        h = jnp.dot(sb, w1_ref[...],
                    preferred_element_type=jnp.float32) + bb1_ref[...]
        hn = (h - st1_ref[0:1, :]) * jax.lax.rsqrt(st1_ref[1:2, :] + EPS)
        hn = hn * g_ref[...] + b_ref[...]
        hg = 0.5 * hn * (1.0 + jax.lax.erf(hn * (2.0 ** -0.5)))
        o = jnp.dot(hg.astype(jnp.bfloat16), w2_ref[...],
                    preferred_element_type=jnp.float32) + bb2_ref[...]
        o2_scr[pl.ds(i * mb, mb), :] = o
        acc_ref[0:1, :] += jnp.sum(o, axis=0, keepdims=True)
        acc_ref[1:2, :] += jnp.sum(o * o, axis=0, keepdims=True)

    @pl.when(ph == 1)
    def _():
        m = acc_ref[0:1, :] / M
        var = acc_ref[1:2, :] / M - m * m
        o = o2_scr[pl.ds(i * mb, mb), :]
        on = (o - m) * jax.lax.rsqrt(var + EPS)
        out_ref[...] = x2_ref[...] + on * g2_ref[...] + b2_ref[...]


def kernel(x, g1, b1, Wqkv, Wo, g2, b2, W1, bb1, gf1, bf1, W2, bb2, gf2, bf2):
    T, B, C, Lt, Lh, Lw = x.shape
    nt, nh, nw = NWIN
    wt, wh, ww = Lt // nt, Lh // nh, Lw // nw
    P = nt * nh * nw
    w = wt * wh * ww
    N = B * P * w
    M = T * N
    Cf = 4 * C
    cb = 128
    ncb = C // cb

    xw = x.transpose(0, 1, 3, 4, 5, 2)
    xw = xw.reshape(T, B, nt, wt, nh, wh, nw, ww, C)
    xw = xw.transpose(0, 1, 2, 4, 6, 3, 5, 7, 8)
    Xw = xw.reshape(T, N, C)

    g1r = g1.reshape(1, 1, C)
    b1r = b1.reshape(1, 1, C)
    g2r = g2.reshape(1, 1, C)
    b2r = b2.reshape(1, 1, C)

    s1 = pl.pallas_call(
        functools.partial(_bnlif_kernel, nck=8),
        grid=(ncb,),
        in_specs=[
            pl.BlockSpec((T, N, cb), lambda i: (0, 0, i)),
            pl.BlockSpec((1, 1, cb), lambda i: (0, 0, i)),
            pl.BlockSpec((1, 1, cb), lambda i: (0, 0, i)),
        ],
        out_specs=pl.BlockSpec((T, N, cb), lambda i: (0, 0, i)),
        out_shape=jax.ShapeDtypeStruct((T, N, C), jnp.bfloat16),
    )(Xw, g1r, b1r)

    TB = T * B
    Pw = P * w
    s1b = s1.reshape(T, B, Pw, C).reshape(TB, Pw, C)
    q, kt, v, idx = pl.pallas_call(
        functools.partial(_route_kernel, P=P, w=w, C=C),
        grid=(TB,),
        in_specs=[
            pl.BlockSpec((1, Pw, C), lambda i: (i, 0, 0)),
            pl.BlockSpec((C, 3 * C), lambda i: (0, 0)),
        ],
        out_specs=[
            pl.BlockSpec((1, Pw, C), lambda i: (i, 0, 0)),
            pl.BlockSpec((1, P, C, w), lambda i: (i, 0, 0, 0)),
            pl.BlockSpec((1, Pw, C), lambda i: (i, 0, 0)),
            pl.BlockSpec((1, K_TOP, P), lambda i: (i, 0, 0)),
        ],
        out_shape=[
            jax.ShapeDtypeStruct((TB, Pw, C), jnp.bfloat16),
            jax.ShapeDtypeStruct((TB, P, C, w), jnp.bfloat16),
            jax.ShapeDtypeStruct((TB, Pw, C), jnp.bfloat16),
            jax.ShapeDtypeStruct((TB, K_TOP, P), jnp.int32),
        ],
    )(s1b, Wqkv)

    G = 8
    scr = ([pltpu.VMEM((C, K_TOP * w), jnp.bfloat16) for _ in range(G)]
           + [pltpu.VMEM((K_TOP * w, C), jnp.bfloat16) for _ in range(G)]
           + [pltpu.VMEM((N_HEADS * w, C), jnp.bfloat16) for _ in range(G)])
    Xwb = Xw.reshape(T, B, Pw, C).reshape(TB, Pw, C)
    x2b = pl.pallas_call(
        functools.partial(_attn_kernel, w=w, C=C, G=G),
        grid_spec=pltpu.PrefetchScalarGridSpec(
            num_scalar_prefetch=1,
            grid=(TB, P // G),
            in_specs=[
                pl.BlockSpec((1, G * w, C), lambda b, pg, idx_ref: (b, pg, 0)),
                pl.BlockSpec((1, P, C, w),
                             lambda b, pg, idx_ref: (b, 0, 0, 0)),
                pl.BlockSpec((1, Pw, C), lambda b, pg, idx_ref: (b, 0, 0)),
                pl.BlockSpec((C, C), lambda b, pg, idx_ref: (0, 0)),
                pl.BlockSpec((1, G * w, C), lambda b, pg, idx_ref: (b, pg, 0)),
            ],
            out_specs=pl.BlockSpec((1, G * w, C),
                                   lambda b, pg, idx_ref: (b, pg, 0)),
            scratch_shapes=scr,
        ),
        out_shape=jax.ShapeDtypeStruct((TB, Pw, C), jnp.float32),
    )(idx, q, kt, v, Wo, Xwb)

    x2 = x2b.reshape(T, N, C)
    W1T = W1.T
    W2Tb = W2.T.astype(jnp.bfloat16)
    W1Tb = W1T.astype(jnp.bfloat16)
    nb = 1024
    nnb = N // nb
    s2, st1 = pl.pallas_call(
        functools.partial(_bnlif_gram_kernel, M=float(M)),
        grid=(2, nnb),
        in_specs=[
            pl.BlockSpec((T, nb, C), lambda ph, i: (0, i, 0)),
            pl.BlockSpec((1, C), lambda ph, i: (0, 0)),
            pl.BlockSpec((1, C), lambda ph, i: (0, 0)),
            pl.BlockSpec((C, Cf), lambda ph, i: (0, 0)),
            pl.BlockSpec((1, Cf), lambda ph, i: (0, 0)),
        ],
        out_specs=[
            pl.BlockSpec((T, nb, C), lambda ph, i: (0, i * ph, 0)),
            pl.BlockSpec((2, Cf), lambda ph, i: (0, 0)),
        ],
        out_shape=[
            jax.ShapeDtypeStruct((T, N, C), jnp.bfloat16),
            jax.ShapeDtypeStruct((2, Cf), jnp.float32),
        ],
        scratch_shapes=[pltpu.VMEM((2, C), jnp.float32),
                        pltpu.VMEM((1, C), jnp.float32),
                        pltpu.VMEM((C, C), jnp.float32)],
    )(x2, g2.reshape(1, C), b2.reshape(1, C), W1T, bb1.reshape(1, Cf))

    s2v = s2.reshape(M, C)
    x2v = x2.reshape(M, C)
    mb = 1024
    nmb = M // mb

    outv = pl.pallas_call(
        functools.partial(_ffn_kernel, M=float(M), mb=mb),
        grid=(2, nmb),
        in_specs=[
            pl.BlockSpec((mb, C), lambda ph, i: (i * (1 - ph), 0)),
            pl.BlockSpec((C, Cf), lambda ph, i: (0, 0)),
            pl.BlockSpec((1, Cf), lambda ph, i: (0, 0)),
            pl.BlockSpec((2, Cf), lambda ph, i: (0, 0)),
            pl.BlockSpec((1, Cf), lambda ph, i: (0, 0)),
            pl.BlockSpec((1, Cf), lambda ph, i: (0, 0)),
            pl.BlockSpec((Cf, C), lambda ph, i: (0, 0)),
            pl.BlockSpec((1, C), lambda ph, i: (0, 0)),
            pl.BlockSpec((mb, C), lambda ph, i: (i * ph, 0)),
            pl.BlockSpec((1, C), lambda ph, i: (0, 0)),
            pl.BlockSpec((1, C), lambda ph, i: (0, 0)),
        ],
        out_specs=pl.BlockSpec((mb, C), lambda ph, i: (i * ph, 0)),
        out_shape=jax.ShapeDtypeStruct((M, C), jnp.float32),
        scratch_shapes=[pltpu.VMEM((2, C), jnp.float32),
                        pltpu.VMEM((M, C), jnp.float32)],
    )(s2v, W1Tb, bb1.reshape(1, Cf), st1, gf1.reshape(1, Cf),
      bf1.reshape(1, Cf), W2Tb, bb2.reshape(1, C), x2v,
      gf2.reshape(1, C), bf2.reshape(1, C))

    out = outv.reshape(T, B, nt, nh, nw, wt, wh, ww, C)
    out = out.transpose(0, 1, 2, 5, 3, 6, 4, 7, 8)
    out = out.reshape(T, B, Lt, Lh, Lw, C)
    return out.transpose(0, 1, 5, 2, 3, 4)


# speedup vs baseline: 3.1775x; 1.0681x over previous
import functools

import jax
import jax.numpy as jnp
from jax.experimental import pallas as pl
from jax.experimental.pallas import tpu as pltpu

TAU = 2.0
THR = 1.0
NWIN = (2, 4, 4)
K_TOP = 4
N_HEADS = 8
EPS = 1e-5


def _bnlif_kernel(x_ref, g_ref, b_ref, s_ref, *, nck):
    T, N, cb = x_ref.shape
    ck = N // nck
    acc = jnp.zeros((1, cb), jnp.float32)
    acc2 = jnp.zeros((1, cb), jnp.float32)
    for c in range(nck):
        xc = x_ref[:, c * ck:(c + 1) * ck, :]
        acc = acc + jnp.sum(xc, axis=(0, 1), keepdims=False)[None]
        acc2 = acc2 + jnp.sum(xc * xc, axis=(0, 1), keepdims=False)[None]
    cnt = float(T * N)
    m = acc / cnt
    var = acc2 / cnt - m * m
    scale = jax.lax.rsqrt(var + EPS) * g_ref[0]
    shift = b_ref[0] - m * scale
    for c in range(nck):
        sl = pl.ds(c * ck, ck)
        v = jnp.zeros((ck, cb), jnp.float32)
        for t in range(T):
            xn = x_ref[t, sl, :] * scale + shift
            v = v + (xn - v) / TAU
            sp = (v - THR >= 0).astype(jnp.float32)
            s_ref[t, sl, :] = sp.astype(jnp.bfloat16)
            v = v - sp * THR


def _add_kernel(x_ref, a_ref, o_ref):
    o_ref[...] = x_ref[...] + a_ref[...]


def _route_kernel(s_ref, wqkv_ref, q_out_ref, kt_ref, v_out_ref, idx_ref,
                  *, P, w, C):
    s = s_ref[0].astype(jnp.float32)
    qkv = jnp.dot(s, wqkv_ref[...], preferred_element_type=jnp.float32)
    q = qkv[:, :C]
    k = qkv[:, C:2 * C]
    q_out_ref[0] = q.astype(jnp.bfloat16)
    v_out_ref[0] = qkv[:, 2 * C:].astype(jnp.bfloat16)
    kt_ref[0] = jnp.swapaxes(k.reshape(P, w, C), 1, 2).astype(jnp.bfloat16)
    qm = jnp.mean(q.reshape(P, w, C), axis=1)
    km = jnp.mean(k.reshape(P, w, C), axis=1)
    aff = jax.lax.dot_general(qm, km, (((1,), (1,)), ((), ())),
                              preferred_element_type=jnp.float32)
    col = jax.lax.broadcasted_iota(jnp.int32, (P, P), 1)
    cur = aff
    for j in range(K_TOP):
        mx = jnp.max(cur, axis=1, keepdims=True)
        cand = jnp.where(cur == mx, col, jnp.int32(2 ** 30))
        idxj = jnp.min(cand, axis=1)
        idx_ref[0, j, :] = idxj
        cur = jnp.where(col == idxj[:, None], -jnp.inf, cur)


def _attn_kernel(idx_sref, q_ref, kt_ref, v_ref, wo_ref, xw_ref, x2_ref,
                 *scr, w, C, G):
    b = pl.program_id(0)
    pg = pl.program_id(1)
    H = N_HEADS
    dh = C // H
    scale = dh ** -0.5
    kt_scr = scr[:G]
    v_scr = scr[G:2 * G]
    qbd_scr = scr[2 * G:]
    lane = jax.lax.broadcasted_iota(jnp.int32, (1, C), 1)
    masks = [((lane >= h * dh) & (lane < (h + 1) * dh)).astype(jnp.bfloat16)
             for h in range(H)]
    for u in range(G):
        p = pg * G + u
        for j in range(K_TOP):
            wi = idx_sref[b, j, p]
            kt_scr[u][:, j * w:(j + 1) * w] = kt_ref[0, wi, :, :]
            v_scr[u][pl.ds(j * w, w), :] = v_ref[0, pl.ds(wi * w, w), :]
        qu = q_ref[0, u * w:(u + 1) * w, :]
        for h in range(H):
            qbd_scr[u][h * w:(h + 1) * w, :] = qu * masks[h]
    outs_w = []
    for u in range(G):
        logits = jnp.dot(qbd_scr[u][...], kt_scr[u][...],
                         preferred_element_type=jnp.float32) * scale
        logits = logits - jnp.max(logits, axis=1, keepdims=True)
        e = jnp.exp(logits)
        att = e / jnp.sum(e, axis=1, keepdims=True)
        ov = jnp.dot(att.astype(jnp.bfloat16), v_scr[u][...],
                     preferred_element_type=jnp.float32)
        o_u = ov[0:w, :] * masks[0]
        for h in range(1, H):
            o_u = o_u + ov[h * w:(h + 1) * w, :] * masks[h]
        outs_w.append(o_u)
    o_all = jnp.concatenate(outs_w, axis=0)
    x2_ref[0] = xw_ref[0] + jnp.dot(o_all, wo_ref[...],
                                    preferred_element_type=jnp.float32)


def _bnlif_gram_kernel(x_ref, g_ref, b_ref, w1_ref, bb_ref, s_ref, st_ref,
                       acc_ref, cs_ref, gram_ref, *, M):
    ph = pl.program_id(0)
    i = pl.program_id(1)
    T, nb, C = x_ref.shape

    @pl.when(ph == 0)
    def _():
        @pl.when(i == 0)
        def _():
            acc_ref[...] = jnp.zeros_like(acc_ref)

        x = x_ref[...]
        acc_ref[0:1, :] += jnp.sum(x, axis=(0, 1), keepdims=False)[None]
        acc_ref[1:2, :] += jnp.sum(x * x, axis=(0, 1), keepdims=False)[None]

    @pl.when(ph == 1)
    def _():
        @pl.when(i == 0)
        def _():
            cs_ref[...] = jnp.zeros_like(cs_ref)
            gram_ref[...] = jnp.zeros_like(gram_ref)

        m = acc_ref[0:1, :] / M
        var = acc_ref[1:2, :] / M - m * m
        scale = jax.lax.rsqrt(var + EPS) * g_ref[...]
        shift = b_ref[...] - m * scale
        v = jnp.zeros((nb, C), jnp.float32)
        sts = []
        for t in range(T):
            xn = x_ref[t] * scale + shift
            v = v + (xn - v) / TAU
            sp = (v - THR >= 0).astype(jnp.float32)
            s_ref[t] = sp.astype(jnp.bfloat16)
            sts.append(sp)
            v = v - sp * THR
        sall = jnp.concatenate(sts, axis=0).astype(jnp.bfloat16)
        cs_ref[...] += jnp.sum(sall.astype(jnp.float32), axis=0,
                               keepdims=True)
        gram_ref[...] += jax.lax.dot_general(
            sall, sall, (((0,), (0,)), ((), ())),
            preferred_element_type=jnp.float32)

        @pl.when(i == pl.num_programs(1) - 1)
        def _():
            wmat = w1_ref[...]
            sw = jnp.dot(gram_ref[...], wmat,
                         preferred_element_type=jnp.float32)
            diag = jnp.sum(wmat * sw, axis=0, keepdims=True)
            cw = jnp.dot(cs_ref[...], wmat, preferred_element_type=jnp.float32)
            bb = bb_ref[...]
            mh = (cw + M * bb) / M
            varh = (diag + 2.0 * bb * cw + M * bb * bb) / M - mh * mh
            st_ref[...] = jnp.concatenate([mh, varh], axis=0)


def _ffn_kernel(s_ref, w1_ref, bb1_ref, st1_ref, g_ref, b_ref,
                w2_ref, bb2_ref, x2_ref, g2_ref, b2_ref, out_ref,
                acc_ref, o2_scr, *, M, mb):
    ph = pl.program_id(0)
    i = pl.program_id(1)

    @pl.when(ph == 0)
    def _():
        @pl.when(i == 0)
        def _():
            acc_ref[...] = jnp.zeros_like(acc_ref)

        sb = s_ref[...]
        h = jnp.dot(sb, w1_ref[...],
                    preferred_element_type=jnp.float32) + bb1_ref[...]
        hn = (h - st1_ref[0:1, :]) * jax.lax.rsqrt(st1_ref[1:2, :] + EPS)
        hn = hn * g_ref[...] + b_ref[...]
        hg = 0.5 * hn * (1.0 + jax.lax.erf(hn * (2.0 ** -0.5)))
        o = jnp.dot(hg.astype(jnp.bfloat16), w2_ref[...],
                    preferred_element_type=jnp.float32) + bb2_ref[...]
        o2_scr[pl.ds(i * mb, mb), :] = o
        acc_ref[0:1, :] += jnp.sum(o, axis=0, keepdims=True)
        acc_ref[1:2, :] += jnp.sum(o * o, axis=0, keepdims=True)

    @pl.when(ph == 1)
    def _():
        m = acc_ref[0:1, :] / M
        var = acc_ref[1:2, :] / M - m * m
        o = o2_scr[pl.ds(i * mb, mb), :]
        on = (o - m) * jax.lax.rsqrt(var + EPS)
        out_ref[...] = x2_ref[...] + on * g2_ref[...] + b2_ref[...]


def kernel(x, g1, b1, Wqkv, Wo, g2, b2, W1, bb1, gf1, bf1, W2, bb2, gf2, bf2):
    T, B, C, Lt, Lh, Lw = x.shape
    nt, nh, nw = NWIN
    wt, wh, ww = Lt // nt, Lh // nh, Lw // nw
    P = nt * nh * nw
    w = wt * wh * ww
    N = B * P * w
    M = T * N
    Cf = 4 * C
    cb = 128
    ncb = C // cb

    xw = x.transpose(0, 1, 3, 4, 5, 2)
    xw = xw.reshape(T, B, nt, wt, nh, wh, nw, ww, C)
    xw = xw.transpose(0, 1, 2, 4, 6, 3, 5, 7, 8)
    Xw = xw.reshape(T, N, C)

    g1r = g1.reshape(1, 1, C)
    b1r = b1.reshape(1, 1, C)
    g2r = g2.reshape(1, 1, C)
    b2r = b2.reshape(1, 1, C)

    s1 = pl.pallas_call(
        functools.partial(_bnlif_kernel, nck=8),
        grid=(ncb,),
        in_specs=[
            pl.BlockSpec((T, N, cb), lambda i: (0, 0, i)),
            pl.BlockSpec((1, 1, cb), lambda i: (0, 0, i)),
            pl.BlockSpec((1, 1, cb), lambda i: (0, 0, i)),
        ],
        out_specs=pl.BlockSpec((T, N, cb), lambda i: (0, 0, i)),
        out_shape=jax.ShapeDtypeStruct((T, N, C), jnp.bfloat16),
    )(Xw, g1r, b1r)

    TB = T * B
    Pw = P * w
    s1b = s1.reshape(T, B, Pw, C).reshape(TB, Pw, C)
    q, kt, v, idx = pl.pallas_call(
        functools.partial(_route_kernel, P=P, w=w, C=C),
        grid=(TB,),
        in_specs=[
            pl.BlockSpec((1, Pw, C), lambda i: (i, 0, 0)),
            pl.BlockSpec((C, 3 * C), lambda i: (0, 0)),
        ],
        out_specs=[
            pl.BlockSpec((1, Pw, C), lambda i: (i, 0, 0)),
            pl.BlockSpec((1, P, C, w), lambda i: (i, 0, 0, 0)),
            pl.BlockSpec((1, Pw, C), lambda i: (i, 0, 0)),
            pl.BlockSpec((1, K_TOP, P), lambda i: (i, 0, 0)),
        ],
        out_shape=[
            jax.ShapeDtypeStruct((TB, Pw, C), jnp.bfloat16),
            jax.ShapeDtypeStruct((TB, P, C, w), jnp.bfloat16),
            jax.ShapeDtypeStruct((TB, Pw, C), jnp.bfloat16),
            jax.ShapeDtypeStruct((TB, K_TOP, P), jnp.int32),
        ],
    )(s1b, Wqkv)

    G = 16
    scr = ([pltpu.VMEM((C, K_TOP * w), jnp.bfloat16) for _ in range(G)]
           + [pltpu.VMEM((K_TOP * w, C), jnp.bfloat16) for _ in range(G)]
           + [pltpu.VMEM((N_HEADS * w, C), jnp.bfloat16) for _ in range(G)])
    Xwb = Xw.reshape(T, B, Pw, C).reshape(TB, Pw, C)
    x2b = pl.pallas_call(
        functools.partial(_attn_kernel, w=w, C=C, G=G),
        grid_spec=pltpu.PrefetchScalarGridSpec(
            num_scalar_prefetch=1,
            grid=(TB, P // G),
            in_specs=[
                pl.BlockSpec((1, G * w, C), lambda b, pg, idx_ref: (b, pg, 0)),
                pl.BlockSpec((1, P, C, w),
                             lambda b, pg, idx_ref: (b, 0, 0, 0)),
                pl.BlockSpec((1, Pw, C), lambda b, pg, idx_ref: (b, 0, 0)),
                pl.BlockSpec((C, C), lambda b, pg, idx_ref: (0, 0)),
                pl.BlockSpec((1, G * w, C), lambda b, pg, idx_ref: (b, pg, 0)),
            ],
            out_specs=pl.BlockSpec((1, G * w, C),
                                   lambda b, pg, idx_ref: (b, pg, 0)),
            scratch_shapes=scr,
        ),
        out_shape=jax.ShapeDtypeStruct((TB, Pw, C), jnp.float32),
    )(idx, q, kt, v, Wo, Xwb)

    x2 = x2b.reshape(T, N, C)
    W1T = W1.T
    W2Tb = W2.T.astype(jnp.bfloat16)
    W1Tb = W1T.astype(jnp.bfloat16)
    nb = 1024
    nnb = N // nb
    s2, st1 = pl.pallas_call(
        functools.partial(_bnlif_gram_kernel, M=float(M)),
        grid=(2, nnb),
        in_specs=[
            pl.BlockSpec((T, nb, C), lambda ph, i: (0, i, 0)),
            pl.BlockSpec((1, C), lambda ph, i: (0, 0)),
            pl.BlockSpec((1, C), lambda ph, i: (0, 0)),
            pl.BlockSpec((C, Cf), lambda ph, i: (0, 0)),
            pl.BlockSpec((1, Cf), lambda ph, i: (0, 0)),
        ],
        out_specs=[
            pl.BlockSpec((T, nb, C), lambda ph, i: (0, i * ph, 0)),
            pl.BlockSpec((2, Cf), lambda ph, i: (0, 0)),
        ],
        out_shape=[
            jax.ShapeDtypeStruct((T, N, C), jnp.bfloat16),
            jax.ShapeDtypeStruct((2, Cf), jnp.float32),
        ],
        scratch_shapes=[pltpu.VMEM((2, C), jnp.float32),
                        pltpu.VMEM((1, C), jnp.float32),
                        pltpu.VMEM((C, C), jnp.float32)],
    )(x2, g2.reshape(1, C), b2.reshape(1, C), W1T, bb1.reshape(1, Cf))

    s2v = s2.reshape(M, C)
    x2v = x2.reshape(M, C)
    mb = 1024
    nmb = M // mb

    outv = pl.pallas_call(
        functools.partial(_ffn_kernel, M=float(M), mb=mb),
        grid=(2, nmb),
        in_specs=[
            pl.BlockSpec((mb, C), lambda ph, i: (i * (1 - ph), 0)),
            pl.BlockSpec((C, Cf), lambda ph, i: (0, 0)),
            pl.BlockSpec((1, Cf), lambda ph, i: (0, 0)),
            pl.BlockSpec((2, Cf), lambda ph, i: (0, 0)),
            pl.BlockSpec((1, Cf), lambda ph, i: (0, 0)),
            pl.BlockSpec((1, Cf), lambda ph, i: (0, 0)),
            pl.BlockSpec((Cf, C), lambda ph, i: (0, 0)),
            pl.BlockSpec((1, C), lambda ph, i: (0, 0)),
            pl.BlockSpec((mb, C), lambda ph, i: (i * ph, 0)),
            pl.BlockSpec((1, C), lambda ph, i: (0, 0)),
            pl.BlockSpec((1, C), lambda ph, i: (0, 0)),
        ],
        out_specs=pl.BlockSpec((mb, C), lambda ph, i: (i * ph, 0)),
        out_shape=jax.ShapeDtypeStruct((M, C), jnp.float32),
        scratch_shapes=[pltpu.VMEM((2, C), jnp.float32),
                        pltpu.VMEM((M, C), jnp.float32)],
    )(s2v, W1Tb, bb1.reshape(1, Cf), st1, gf1.reshape(1, Cf),
      bf1.reshape(1, Cf), W2Tb, bb2.reshape(1, C), x2v,
      gf2.reshape(1, C), bf2.reshape(1, C))

    out = outv.reshape(T, B, nt, nh, nw, wt, wh, ww, C)
    out = out.transpose(0, 1, 2, 5, 3, 6, 4, 7, 8)
    out = out.reshape(T, B, Lt, Lh, Lw, C)
    return out.transpose(0, 1, 5, 2, 3, 4)


# speedup vs baseline: 3.2346x; 1.0180x over previous
import functools

import jax
import jax.numpy as jnp
from jax.experimental import pallas as pl
from jax.experimental.pallas import tpu as pltpu

TAU = 2.0
THR = 1.0
NWIN = (2, 4, 4)
K_TOP = 4
N_HEADS = 8
EPS = 1e-5


def _bnlif_kernel(x_ref, g_ref, b_ref, s_ref, *, nck):
    T, N, cb = x_ref.shape
    ck = N // nck
    acc = jnp.zeros((1, cb), jnp.float32)
    acc2 = jnp.zeros((1, cb), jnp.float32)
    for c in range(nck):
        xc = x_ref[:, c * ck:(c + 1) * ck, :]
        acc = acc + jnp.sum(xc, axis=(0, 1), keepdims=False)[None]
        acc2 = acc2 + jnp.sum(xc * xc, axis=(0, 1), keepdims=False)[None]
    cnt = float(T * N)
    m = acc / cnt
    var = acc2 / cnt - m * m
    scale = jax.lax.rsqrt(var + EPS) * g_ref[0]
    shift = b_ref[0] - m * scale
    for c in range(nck):
        sl = pl.ds(c * ck, ck)
        v = jnp.zeros((ck, cb), jnp.float32)
        for t in range(T):
            xn = x_ref[t, sl, :] * scale + shift
            v = v + (xn - v) / TAU
            sp = (v - THR >= 0).astype(jnp.float32)
            s_ref[t, sl, :] = sp.astype(jnp.bfloat16)
            v = v - sp * THR


def _add_kernel(x_ref, a_ref, o_ref):
    o_ref[...] = x_ref[...] + a_ref[...]


def _route_kernel(s_ref, wqkv_ref, q_out_ref, kt_ref, v_out_ref, idx_ref,
                  *, P, w, C):
    s = s_ref[0].astype(jnp.float32)
    qkv = jnp.dot(s, wqkv_ref[...], preferred_element_type=jnp.float32)
    q = qkv[:, :C]
    k = qkv[:, C:2 * C]
    q_out_ref[0] = q.astype(jnp.bfloat16)
    v_out_ref[0] = qkv[:, 2 * C:].astype(jnp.bfloat16)
    kt_ref[0] = jnp.swapaxes(k.reshape(P, w, C), 1, 2).astype(jnp.bfloat16)
    qm = jnp.mean(q.reshape(P, w, C), axis=1)
    km = jnp.mean(k.reshape(P, w, C), axis=1)
    aff = jax.lax.dot_general(qm, km, (((1,), (1,)), ((), ())),
                              preferred_element_type=jnp.float32)
    col = jax.lax.broadcasted_iota(jnp.int32, (P, P), 1)
    cur = aff
    for j in range(K_TOP):
        mx = jnp.max(cur, axis=1, keepdims=True)
        cand = jnp.where(cur == mx, col, jnp.int32(2 ** 30))
        idxj = jnp.min(cand, axis=1)
        idx_ref[0, j, :] = idxj
        cur = jnp.where(col == idxj[:, None], -jnp.inf, cur)


def _attn_kernel(idx_sref, q_ref, kt_ref, v_ref, wo_ref, xw_ref, x2_ref,
                 *scr, w, C, G):
    b = pl.program_id(0)
    pg = pl.program_id(1)
    H = N_HEADS
    dh = C // H
    scale = dh ** -0.5
    kt_scr = scr[:G]
    v_scr = scr[G:2 * G]
    qbd_scr = scr[2 * G:]
    lane = jax.lax.broadcasted_iota(jnp.int32, (1, C), 1)
    masks = [((lane >= h * dh) & (lane < (h + 1) * dh)).astype(jnp.bfloat16)
             for h in range(H)]
    for u in range(G):
        p = pg * G + u
        for j in range(K_TOP):
            wi = idx_sref[b, j, p]
            kt_scr[u][:, j * w:(j + 1) * w] = kt_ref[0, wi, :, :]
            v_scr[u][pl.ds(j * w, w), :] = v_ref[0, pl.ds(wi * w, w), :]
        qu = q_ref[0, u * w:(u + 1) * w, :]
        for h in range(H):
            qbd_scr[u][h * w:(h + 1) * w, :] = qu * masks[h]
    outs_w = []
    for u in range(G):
        logits = jnp.dot(qbd_scr[u][...], kt_scr[u][...],
                         preferred_element_type=jnp.float32) * scale
        logits = logits - jnp.max(logits, axis=1, keepdims=True)
        e = jnp.exp(logits)
        att = e / jnp.sum(e, axis=1, keepdims=True)
        ov = jnp.dot(att.astype(jnp.bfloat16), v_scr[u][...],
                     preferred_element_type=jnp.float32)
        o_u = ov[0:w, :] * masks[0]
        for h in range(1, H):
            o_u = o_u + ov[h * w:(h + 1) * w, :] * masks[h]
        outs_w.append(o_u)
    o_all = jnp.concatenate(outs_w, axis=0)
    x2_ref[0] = xw_ref[0] + jnp.dot(o_all, wo_ref[...],
                                    preferred_element_type=jnp.float32)


def _bnlif_gram_kernel(x_ref, g_ref, b_ref, w1_ref, bb_ref, s_ref, st_ref,
                       acc_ref, cs_ref, gram_ref, *, M):
    ph = pl.program_id(0)
    i = pl.program_id(1)
    T, nb, C = x_ref.shape

    @pl.when(ph == 0)
    def _():
        @pl.when(i == 0)
        def _():
            acc_ref[...] = jnp.zeros_like(acc_ref)

        x = x_ref[...]
        acc_ref[0:1, :] += jnp.sum(x, axis=(0, 1), keepdims=False)[None]
        acc_ref[1:2, :] += jnp.sum(x * x, axis=(0, 1), keepdims=False)[None]

    @pl.when(ph == 1)
    def _():
        @pl.when(i == 0)
        def _():
            cs_ref[...] = jnp.zeros_like(cs_ref)
            gram_ref[...] = jnp.zeros_like(gram_ref)

        m = acc_ref[0:1, :] / M
        var = acc_ref[1:2, :] / M - m * m
        scale = jax.lax.rsqrt(var + EPS) * g_ref[...]
        shift = b_ref[...] - m * scale
        v = jnp.zeros((nb, C), jnp.float32)
        sts = []
        for t in range(T):
            xn = x_ref[t] * scale + shift
            v = v + (xn - v) / TAU
            sp = (v - THR >= 0).astype(jnp.float32)
            s_ref[t] = sp.astype(jnp.bfloat16)
            sts.append(sp)
            v = v - sp * THR
        sall = jnp.concatenate(sts, axis=0).astype(jnp.bfloat16)
        cs_ref[...] += jnp.sum(sall.astype(jnp.float32), axis=0,
                               keepdims=True)
        gram_ref[...] += jax.lax.dot_general(
            sall, sall, (((0,), (0,)), ((), ())),
            preferred_element_type=jnp.float32)

        @pl.when(i == pl.num_programs(1) - 1)
        def _():
            wmat = w1_ref[...]
            sw = jnp.dot(gram_ref[...], wmat,
                         preferred_element_type=jnp.float32)
            diag = jnp.sum(wmat * sw, axis=0, keepdims=True)
            cw = jnp.dot(cs_ref[...], wmat, preferred_element_type=jnp.float32)
            bb = bb_ref[...]
            mh = (cw + M * bb) / M
            varh = (diag + 2.0 * bb * cw + M * bb * bb) / M - mh * mh
            st_ref[...] = jnp.concatenate([mh, varh], axis=0)


def _ffn_kernel(s_ref, w1_ref, bb1_ref, st1_ref, g_ref, b_ref,
                w2_ref, bb2_ref, x2_ref, g2_ref, b2_ref, out_ref,
                acc_ref, o2_scr, *, M, mb):
    ph = pl.program_id(0)
    i = pl.program_id(1)

    @pl.when(ph == 0)
    def _():
        @pl.when(i == 0)
        def _():
            acc_ref[...] = jnp.zeros_like(acc_ref)

        sb = s_ref[...]
        h = jnp.dot(sb, w1_ref[...],
                    preferred_element_type=jnp.float32) + bb1_ref[...]
        hn = (h - st1_ref[0:1, :]) * jax.lax.rsqrt(st1_ref[1:2, :] + EPS)
        hn = hn * g_ref[...] + b_ref[...]
        hg = 0.5 * hn * (1.0 + jax.lax.erf(hn * (2.0 ** -0.5)))
        o = jnp.dot(hg.astype(jnp.bfloat16), w2_ref[...],
                    preferred_element_type=jnp.float32) + bb2_ref[...]
        o2_scr[pl.ds(i * mb, mb), :] = o
        acc_ref[0:1, :] += jnp.sum(o, axis=0, keepdims=True)
        acc_ref[1:2, :] += jnp.sum(o * o, axis=0, keepdims=True)

    @pl.when(ph == 1)
    def _():
        m = acc_ref[0:1, :] / M
        var = acc_ref[1:2, :] / M - m * m
        o = o2_scr[pl.ds(i * mb, mb), :]
        on = (o - m) * jax.lax.rsqrt(var + EPS)
        out_ref[...] = x2_ref[...] + on * g2_ref[...] + b2_ref[...]


def kernel(x, g1, b1, Wqkv, Wo, g2, b2, W1, bb1, gf1, bf1, W2, bb2, gf2, bf2):
    T, B, C, Lt, Lh, Lw = x.shape
    nt, nh, nw = NWIN
    wt, wh, ww = Lt // nt, Lh // nh, Lw // nw
    P = nt * nh * nw
    w = wt * wh * ww
    N = B * P * w
    M = T * N
    Cf = 4 * C
    cb = 128
    ncb = C // cb

    xw = x.transpose(0, 1, 3, 4, 5, 2)
    xw = xw.reshape(T, B, nt, wt, nh, wh, nw, ww, C)
    xw = xw.transpose(0, 1, 2, 4, 6, 3, 5, 7, 8)
    Xw = xw.reshape(T, N, C)

    g1r = g1.reshape(1, 1, C)
    b1r = b1.reshape(1, 1, C)
    g2r = g2.reshape(1, 1, C)
    b2r = b2.reshape(1, 1, C)

    s1 = pl.pallas_call(
        functools.partial(_bnlif_kernel, nck=8),
        grid=(ncb,),
        in_specs=[
            pl.BlockSpec((T, N, cb), lambda i: (0, 0, i)),
            pl.BlockSpec((1, 1, cb), lambda i: (0, 0, i)),
            pl.BlockSpec((1, 1, cb), lambda i: (0, 0, i)),
        ],
        out_specs=pl.BlockSpec((T, N, cb), lambda i: (0, 0, i)),
        out_shape=jax.ShapeDtypeStruct((T, N, C), jnp.bfloat16),
    )(Xw, g1r, b1r)

    TB = T * B
    Pw = P * w
    s1b = s1.reshape(T, B, Pw, C).reshape(TB, Pw, C)
    q, kt, v, idx = pl.pallas_call(
        functools.partial(_route_kernel, P=P, w=w, C=C),
        grid=(TB,),
        in_specs=[
            pl.BlockSpec((1, Pw, C), lambda i: (i, 0, 0)),
            pl.BlockSpec((C, 3 * C), lambda i: (0, 0)),
        ],
        out_specs=[
            pl.BlockSpec((1, Pw, C), lambda i: (i, 0, 0)),
            pl.BlockSpec((1, P, C, w), lambda i: (i, 0, 0, 0)),
            pl.BlockSpec((1, Pw, C), lambda i: (i, 0, 0)),
            pl.BlockSpec((1, K_TOP, P), lambda i: (i, 0, 0)),
        ],
        out_shape=[
            jax.ShapeDtypeStruct((TB, Pw, C), jnp.bfloat16),
            jax.ShapeDtypeStruct((TB, P, C, w), jnp.bfloat16),
            jax.ShapeDtypeStruct((TB, Pw, C), jnp.bfloat16),
            jax.ShapeDtypeStruct((TB, K_TOP, P), jnp.int32),
        ],
    )(s1b, Wqkv)

    G = 32
    scr = ([pltpu.VMEM((C, K_TOP * w), jnp.bfloat16) for _ in range(G)]
           + [pltpu.VMEM((K_TOP * w, C), jnp.bfloat16) for _ in range(G)]
           + [pltpu.VMEM((N_HEADS * w, C), jnp.bfloat16) for _ in range(G)])
    Xwb = Xw.reshape(T, B, Pw, C).reshape(TB, Pw, C)
    x2b = pl.pallas_call(
        functools.partial(_attn_kernel, w=w, C=C, G=G),
        grid_spec=pltpu.PrefetchScalarGridSpec(
            num_scalar_prefetch=1,
            grid=(TB, P // G),
            in_specs=[
                pl.BlockSpec((1, G * w, C), lambda b, pg, idx_ref: (b, pg, 0)),
                pl.BlockSpec((1, P, C, w),
                             lambda b, pg, idx_ref: (b, 0, 0, 0)),
                pl.BlockSpec((1, Pw, C), lambda b, pg, idx_ref: (b, 0, 0)),
                pl.BlockSpec((C, C), lambda b, pg, idx_ref: (0, 0)),
                pl.BlockSpec((1, G * w, C), lambda b, pg, idx_ref: (b, pg, 0)),
            ],
            out_specs=pl.BlockSpec((1, G * w, C),
                                   lambda b, pg, idx_ref: (b, pg, 0)),
            scratch_shapes=scr,
        ),
        out_shape=jax.ShapeDtypeStruct((TB, Pw, C), jnp.float32),
    )(idx, q, kt, v, Wo, Xwb)

    x2 = x2b.reshape(T, N, C)
    W1T = W1.T
    W2Tb = W2.T.astype(jnp.bfloat16)
    W1Tb = W1T.astype(jnp.bfloat16)
    nb = 1024
    nnb = N // nb
    s2, st1 = pl.pallas_call(
        functools.partial(_bnlif_gram_kernel, M=float(M)),
        grid=(2, nnb),
        in_specs=[
            pl.BlockSpec((T, nb, C), lambda ph, i: (0, i, 0)),
            pl.BlockSpec((1, C), lambda ph, i: (0, 0)),
            pl.BlockSpec((1, C), lambda ph, i: (0, 0)),
            pl.BlockSpec((C, Cf), lambda ph, i: (0, 0)),
            pl.BlockSpec((1, Cf), lambda ph, i: (0, 0)),
        ],
        out_specs=[
            pl.BlockSpec((T, nb, C), lambda ph, i: (0, i * ph, 0)),
            pl.BlockSpec((2, Cf), lambda ph, i: (0, 0)),
        ],
        out_shape=[
            jax.ShapeDtypeStruct((T, N, C), jnp.bfloat16),
            jax.ShapeDtypeStruct((2, Cf), jnp.float32),
        ],
        scratch_shapes=[pltpu.VMEM((2, C), jnp.float32),
                        pltpu.VMEM((1, C), jnp.float32),
                        pltpu.VMEM((C, C), jnp.float32)],
    )(x2, g2.reshape(1, C), b2.reshape(1, C), W1T, bb1.reshape(1, Cf))

    s2v = s2.reshape(M, C)
    x2v = x2.reshape(M, C)
    mb = 1024
    nmb = M // mb

    outv = pl.pallas_call(
        functools.partial(_ffn_kernel, M=float(M), mb=mb),
        grid=(2, nmb),
        in_specs=[
            pl.BlockSpec((mb, C), lambda ph, i: (i * (1 - ph), 0)),
            pl.BlockSpec((C, Cf), lambda ph, i: (0, 0)),
            pl.BlockSpec((1, Cf), lambda ph, i: (0, 0)),
            pl.BlockSpec((2, Cf), lambda ph, i: (0, 0)),
            pl.BlockSpec((1, Cf), lambda ph, i: (0, 0)),
            pl.BlockSpec((1, Cf), lambda ph, i: (0, 0)),
            pl.BlockSpec((Cf, C), lambda ph, i: (0, 0)),
            pl.BlockSpec((1, C), lambda ph, i: (0, 0)),
            pl.BlockSpec((mb, C), lambda ph, i: (i * ph, 0)),
            pl.BlockSpec((1, C), lambda ph, i: (0, 0)),
            pl.BlockSpec((1, C), lambda ph, i: (0, 0)),
        ],
        out_specs=pl.BlockSpec((mb, C), lambda ph, i: (i * ph, 0)),
        out_shape=jax.ShapeDtypeStruct((M, C), jnp.float32),
        scratch_shapes=[pltpu.VMEM((2, C), jnp.float32),
                        pltpu.VMEM((M, C), jnp.float32)],
    )(s2v, W1Tb, bb1.reshape(1, Cf), st1, gf1.reshape(1, Cf),
      bf1.reshape(1, Cf), W2Tb, bb2.reshape(1, C), x2v,
      gf2.reshape(1, C), bf2.reshape(1, C))

    out = outv.reshape(T, B, nt, nh, nw, wt, wh, ww, C)
    out = out.transpose(0, 1, 2, 5, 3, 6, 4, 7, 8)
    out = out.reshape(T, B, Lt, Lh, Lw, C)
    return out.transpose(0, 1, 5, 2, 3, 4)
